# Initial kernel scaffold; baseline (speedup 1.0000x reference)
import jax
import jax.numpy as jnp
from jax.experimental import pallas as pl


def kernel(node_attr, edge_index, edge_attr, edge_length, ee_index, ee_angle, W_proj, b_proj, W_bond, b_bond, W_edgefn, b_edgefn, W_bu, b_bu, W_au, b_au, gnn_bias, gru_Wih, gru_Whh, gru_bih, gru_bhh, s2s_Wih0, s2s_Whh0, s2s_bih0, s2s_bhh0, s2s_Wih1, s2s_Whh1, s2s_bih1, s2s_bhh1, W_sp, b_sp, prelu_a):
    raise NotImplementedError("write your pallas kernel here")


# speedup vs baseline: 7.5888x; 7.5888x over previous
import functools

import jax
import jax.numpy as jnp
from jax import lax
from jax.experimental import pallas as pl
from jax.experimental.pallas import tpu as pltpu

N = 20000
E = 50000
E2 = 100000
D_NODE = 110
D_EATTR = 8
K = 8
H = 32
D_HID = 4096
STEPS = 2
POOL_ITERS = 3

CUTOFF = 5.0
GAMMA = 10.0
CENTERS = [CUTOFF * i / (K - 1) for i in range(K)]

NW = 32
CH = 128
C_E = 13
C_E2 = 25
C_G = C_E + C_E2
EP = NW * C_E * CH
E2P = NW * C_E2 * CH
BG = NW * C_G * CH
NP = 20480
BLK = 2048

_INTERP = False


def _rbf_col(d, k):
    return jnp.exp(-GAMMA * (d - CENTERS[k]) ** 2)


def _k0_body(na_ref, wp_ref, bp_ref, o_ref):
    i = pl.program_id(0)
    x = jnp.maximum(jnp.dot(na_ref[...], wp_ref[...],
                            preferred_element_type=jnp.float32) + bp_ref[...], 0.0)
    rows = i * BLK + lax.broadcasted_iota(jnp.int32, (BLK, 1), 0)
    o_ref[...] = jnp.where(rows < N, x, 0.0)


def _node_proj(na_pad, wp_pad, bp):
    return pl.pallas_call(
        _k0_body,
        grid=(NP // BLK,),
        in_specs=[
            pl.BlockSpec((BLK, 112), lambda i: (i, 0)),
            pl.BlockSpec((112, H), lambda i: (0, 0)),
            pl.BlockSpec((1, H), lambda i: (0, 0)),
        ],
        out_specs=pl.BlockSpec((BLK, H), lambda i: (i, 0)),
        out_shape=jax.ShapeDtypeStruct((NP, H), jnp.float32),
        interpret=_INTERP,
    )(na_pad, wp_pad, bp)


def _kb_body(ang_ref, g2_ref, bm2_ref, wau_ref, bau_ref, w2f_ref, bl_ref, o_ref):
    i = pl.program_id(0)
    hl = jnp.maximum(jnp.dot(bm2_ref[...], wau_ref[...],
                             preferred_element_type=jnp.float32) + bau_ref[...], 0.0)
    y = jnp.dot(hl, w2f_ref[...], preferred_element_type=jnp.float32)
    acc = jnp.dot(hl, bl_ref[...], preferred_element_type=jnp.float32)
    ang = ang_ref[...]
    el = g2_ref[:, 0:1]
    for k in range(K):
        acc += y[:, k * H:(k + 1) * H] * _rbf_col(ang, k)
    for k in range(K):
        acc += y[:, (K + k) * H:(K + k + 1) * H] * _rbf_col(el, k)
    rows = i * BLK + lax.broadcasted_iota(jnp.int32, (BLK, 1), 0)
    o_ref[...] = jnp.where(rows < E2, acc, 0.0)


def _line_msg(ang_p, g2, bm2, wau, bau, w2f, bline):
    return pl.pallas_call(
        _kb_body,
        grid=(E2P // BLK,),
        in_specs=[
            pl.BlockSpec((BLK, 1), lambda i: (i, 0)),
            pl.BlockSpec((BLK, 16), lambda i: (i, 0)),
            pl.BlockSpec((BLK, H), lambda i: (i, 0)),
            pl.BlockSpec((H, H), lambda i: (0, 0)),
            pl.BlockSpec((1, H), lambda i: (0, 0)),
            pl.BlockSpec((H, 16 * H), lambda i: (0, 0)),
            pl.BlockSpec((H, H), lambda i: (0, 0)),
        ],
        out_specs=pl.BlockSpec((BLK, H), lambda i: (i, 0)),
        out_shape=jax.ShapeDtypeStruct((E2P, H), jnp.float32),
        interpret=_INTERP,
    )(ang_p, g2, bm2, wau, bau, w2f, bline)


def _kc_body(ea_ref, el_ref, bm_ref, ap_ref, wbu_ref, bbu_ref, wbf_ref, bb_ref, o_ref):
    i = pl.program_id(0)
    hb = jnp.maximum(jnp.dot(bm_ref[...], wbu_ref[...],
                             preferred_element_type=jnp.float32) + bbu_ref[...], 0.0)
    hb = hb + ap_ref[0] + ap_ref[1]
    y = jnp.dot(hb, wbf_ref[...], preferred_element_type=jnp.float32)
    acc = jnp.dot(hb, bb_ref[...], preferred_element_type=jnp.float32)
    ea = ea_ref[...]
    el = el_ref[...]
    for k in range(D_EATTR):
        acc += y[:, k * H:(k + 1) * H] * ea[:, k:k + 1]
    for k in range(K):
        acc += y[:, (D_EATTR + k) * H:(D_EATTR + k + 1) * H] * _rbf_col(el, k)
    rows = i * BLK + lax.broadcasted_iota(jnp.int32, (BLK, 1), 0)
    o_ref[...] = jnp.where(rows < E, acc, 0.0)


def _bond_msg(ea_p, el_p, bm, ap, wbu, bbu, wbf, bbond):
    return pl.pallas_call(
        _kc_body,
        grid=(EP // BLK,),
        in_specs=[
            pl.BlockSpec((BLK, D_EATTR), lambda i: (i, 0)),
            pl.BlockSpec((BLK, 1), lambda i: (i, 0)),
            pl.BlockSpec((BLK, H), lambda i: (i, 0)),
            pl.BlockSpec((2, BLK, H), lambda i: (0, i, 0)),
            pl.BlockSpec((H, H), lambda i: (0, 0)),
            pl.BlockSpec((1, H), lambda i: (0, 0)),
            pl.BlockSpec((H, 16 * H), lambda i: (0, 0)),
            pl.BlockSpec((H, H), lambda i: (0, 0)),
        ],
        out_specs=pl.BlockSpec((BLK, H), lambda i: (i, 0)),
        out_shape=jax.ShapeDtypeStruct((EP, H), jnp.float32),
        interpret=_INTERP,
    )(ea_p, el_p, bm, ap, wbu, bbu, wbf, bbond)


def _kd_body(np_ref, h_ref, gb_ref, wih_ref, whh_ref, bih_ref, bhh_ref, o_ref):
    x = jnp.maximum(np_ref[0] + np_ref[1] + gb_ref[...], 0.0)
    h = h_ref[...]
    gi = jnp.dot(x, wih_ref[...].T, preferred_element_type=jnp.float32) + bih_ref[...]
    gh = jnp.dot(h, whh_ref[...].T, preferred_element_type=jnp.float32) + bhh_ref[...]
    r = jax.nn.sigmoid(gi[:, :H] + gh[:, :H])
    z = jax.nn.sigmoid(gi[:, H:2 * H] + gh[:, H:2 * H])
    n = jnp.tanh(gi[:, 2 * H:] + r * gh[:, 2 * H:])
    o_ref[...] = (1.0 - z) * n + z * h


def _gru_step(npart, h_gru, gnn_bias, wih, whh, bih, bhh):
    return pl.pallas_call(
        _kd_body,
        grid=(NP // BLK,),
        in_specs=[
            pl.BlockSpec((2, BLK, H), lambda i: (0, i, 0)),
            pl.BlockSpec((BLK, H), lambda i: (i, 0)),
            pl.BlockSpec((1, H), lambda i: (0, 0)),
            pl.BlockSpec((3 * H, H), lambda i: (0, 0)),
            pl.BlockSpec((3 * H, H), lambda i: (0, 0)),
            pl.BlockSpec((1, 3 * H), lambda i: (0, 0)),
            pl.BlockSpec((1, 3 * H), lambda i: (0, 0)),
        ],
        out_specs=pl.BlockSpec((BLK, H), lambda i: (i, 0)),
        out_shape=jax.ShapeDtypeStruct((NP, H), jnp.float32),
        interpret=_INTERP,
    )(npart, h_gru, gnn_bias, wih, whh, bih, bhh)


def _lstm(x, h, c, wih, whh, bih, bhh):
    d = h.shape[-1]
    g = (jnp.dot(x, wih.T, preferred_element_type=jnp.float32) + bih
         + jnp.dot(h, whh.T, preferred_element_type=jnp.float32) + bhh)
    i = jax.nn.sigmoid(g[:, :d])
    f = jax.nn.sigmoid(g[:, d:2 * d])
    gg = jnp.tanh(g[:, 2 * d:3 * d])
    o = jax.nn.sigmoid(g[:, 3 * d:])
    c2 = f * c + i * gg
    return o * jnp.tanh(c2), c2


def _kz_body(x_ref, nf_ref, wih0_ref, whh0_ref, bih0_ref, bhh0_ref,
             wih1_ref, whh1_ref, bih1_ref, bhh1_ref, wsp_ref, bsp_ref, pa_ref,
             o_ref):
    na = jnp.concatenate([x_ref[...], nf_ref[...]], axis=1)
    rows = lax.broadcasted_iota(jnp.int32, (NP, 1), 0)
    valid = rows < N
    d = 2 * H
    q_star = jnp.zeros((1, 2 * d), jnp.float32)
    h0 = jnp.zeros((1, d), jnp.float32)
    c0 = jnp.zeros((1, d), jnp.float32)
    h1 = jnp.zeros((1, d), jnp.float32)
    c1 = jnp.zeros((1, d), jnp.float32)
    for _ in range(POOL_ITERS):
        h0, c0 = _lstm(q_star, h0, c0, wih0_ref[...], whh0_ref[...],
                       bih0_ref[...], bhh0_ref[...])
        h1, c1 = _lstm(h0, h1, c1, wih1_ref[...], whh1_ref[...],
                       bih1_ref[...], bhh1_ref[...])
        q = h1
        e = jnp.sum(na * q, axis=-1, keepdims=True)
        e = jnp.where(valid, e, -1e30)
        m = jnp.max(e, axis=0, keepdims=True)
        p = jnp.where(valid, jnp.exp(e - m), 0.0)
        alpha = p / jnp.sum(p, axis=0, keepdims=True)
        readout = jnp.sum(na * alpha, axis=0, keepdims=True)
        q_star = jnp.concatenate([q, readout], axis=-1)
    y = jnp.dot(q_star, wsp_ref[...], preferred_element_type=jnp.float32) + bsp_ref[...]
    o_ref[...] = jnp.where(y >= 0.0, y, pa_ref[...] * y)


def _set2set(x, nf, wih0, whh0, bih0, bhh0, wih1, whh1, bih1, bhh1, wsp, bsp, pa):
    full = lambda s: pl.BlockSpec(s, lambda: tuple(0 for _ in s))
    return pl.pallas_call(
        _kz_body,
        in_specs=[
            full((NP, H)), full((NP, H)),
            full((4 * 2 * H, 4 * H)), full((4 * 2 * H, 2 * H)),
            full((1, 4 * 2 * H)), full((1, 4 * 2 * H)),
            full((4 * 2 * H, 2 * H)), full((4 * 2 * H, 2 * H)),
            full((1, 4 * 2 * H)), full((1, 4 * 2 * H)),
            full((4 * H, D_HID)), full((1, D_HID)), full((1, 1)),
        ],
        out_specs=full((1, D_HID)),
        out_shape=jax.ShapeDtypeStruct((1, D_HID), jnp.float32),
        interpret=_INTERP,
    )(x, nf, wih0, whh0, bih0, bhh0, wih1, whh1, bih1, bhh1, wsp, bsp, pa)


def _gather_rows(table, idx_pad):
    return jnp.take(table, idx_pad, axis=0)


def _scatter_add(payload, idx_pad, nseg):
    seg = jax.ops.segment_sum(payload, idx_pad, num_segments=nseg)
    return jnp.stack([seg, jnp.zeros_like(seg)])


def kernel(node_attr, edge_index, edge_attr, edge_length, ee_index, ee_angle,
           W_proj, b_proj, W_bond, b_bond, W_edgefn, b_edgefn, W_bu, b_bu,
           W_au, b_au, gnn_bias, gru_Wih, gru_Whh, gru_bih, gru_bhh,
           s2s_Wih0, s2s_Whh0, s2s_bih0, s2s_bhh0,
           s2s_Wih1, s2s_Whh1, s2s_bih1, s2s_bhh1, W_sp, b_sp, prelu_a):
    f32 = jnp.float32
    src = edge_index[0].astype(jnp.int32)
    dst = edge_index[1].astype(jnp.int32)
    ee_src = ee_index[0].astype(jnp.int32)
    ee_dst = ee_index[1].astype(jnp.int32)

    na_pad = jnp.pad(node_attr, ((0, NP - N), (0, 112 - D_NODE)))
    wp_pad = jnp.pad(W_proj, ((0, 112 - D_NODE), (0, 0)))
    w2f = W_edgefn.reshape(16, H, H).transpose(1, 0, 2).reshape(H, 16 * H)
    bline = b_edgefn.reshape(H, H)
    wbf = W_bond.reshape(16, H, H).transpose(1, 0, 2).reshape(H, 16 * H)
    bbond = b_bond.reshape(H, H)

    ee_src_p = jnp.pad(ee_src, (0, E2P - E2))
    ee_dst_p = jnp.pad(ee_dst, (0, E2P - E2))
    dst_p = jnp.pad(dst, (0, EP - E))
    src_p = jnp.pad(src, (0, EP - E))
    ang_p = jnp.pad(ee_angle, (0, E2P - E2))[:, None]
    ea_p = jnp.pad(edge_attr, ((0, EP - E), (0, 0)))
    el_p = jnp.pad(edge_length, (0, EP - E))[:, None]

    src_bits = lax.bitcast_convert_type(src, f32)
    table16 = jnp.zeros((E, 16), f32)
    table16 = table16.at[:, 0].set(edge_length)
    table16 = table16.at[:, 1].set(src_bits)

    g2 = _gather_rows(table16, ee_src_p)
    src2_p = lax.bitcast_convert_type(g2[:, 1], jnp.int32)
    idxg = jnp.concatenate([src_p, src2_p])

    nf = _node_proj(na_pad, wp_pad, b_proj[None])

    x = nf
    h_gru = nf
    for _ in range(STEPS):
        g = _gather_rows(x, idxg)
        bm = g[:EP]
        bm2 = g[EP:]
        out_line = _line_msg(ang_p, g2, bm2, W_au, b_au[None], w2f, bline)
        ap = _scatter_add(out_line, ee_dst_p, EP)
        m = _bond_msg(ea_p, el_p, bm, ap, W_bu, b_bu[None], wbf, bbond)
        npart = _scatter_add(m, dst_p, NP)
        x = _gru_step(npart, h_gru, gnn_bias[None], gru_Wih, gru_Whh,
                      gru_bih[None], gru_bhh[None])
        h_gru = x

    return _set2set(x, nf, s2s_Wih0, s2s_Whh0, s2s_bih0[None], s2s_bhh0[None],
                    s2s_Wih1, s2s_Whh1, s2s_bih1[None], s2s_bhh1[None],
                    W_sp, b_sp[None], prelu_a.reshape(1, 1))


# speedup vs baseline: 11.0659x; 1.4582x over previous
import functools

import jax
import jax.numpy as jnp
from jax import lax
from jax.experimental import pallas as pl
from jax.experimental.pallas import tpu as pltpu
from jax.experimental.pallas import tpu_sc as plsc

N = 20000
E = 50000
E2 = 100000
D_NODE = 110
D_EATTR = 8
K = 8
H = 32
D_HID = 4096
STEPS = 2
POOL_ITERS = 3

CUTOFF = 5.0
GAMMA = 10.0
CENTERS = [CUTOFF * i / (K - 1) for i in range(K)]

NW = 32
CH = 128
C_E = 13
C_E2 = 25
C_G = C_E + C_E2
EP = NW * C_E * CH
E2P = NW * C_E2 * CH
BG = NW * C_G * CH
NP = 20480
BLK = 2048

_INTERP = False


def _rbf_col(d, k):
    return jnp.exp(-GAMMA * (d - CENTERS[k]) ** 2)


def _k0_body(na_ref, wp_ref, bp_ref, o_ref):
    i = pl.program_id(0)
    x = jnp.maximum(jnp.dot(na_ref[...], wp_ref[...],
                            preferred_element_type=jnp.float32) + bp_ref[...], 0.0)
    rows = i * BLK + lax.broadcasted_iota(jnp.int32, (BLK, 1), 0)
    o_ref[...] = jnp.where(rows < N, x, 0.0)


def _node_proj(na_pad, wp_pad, bp):
    return pl.pallas_call(
        _k0_body,
        grid=(NP // BLK,),
        in_specs=[
            pl.BlockSpec((BLK, 112), lambda i: (i, 0)),
            pl.BlockSpec((112, H), lambda i: (0, 0)),
            pl.BlockSpec((1, H), lambda i: (0, 0)),
        ],
        out_specs=pl.BlockSpec((BLK, H), lambda i: (i, 0)),
        out_shape=jax.ShapeDtypeStruct((NP, H), jnp.float32),
        interpret=_INTERP,
    )(na_pad, wp_pad, bp)


def _kb_body(ang_ref, g2_ref, bm2_ref, wau_ref, bau_ref, w2f_ref, bl_ref, o_ref):
    i = pl.program_id(0)
    hl = jnp.maximum(jnp.dot(bm2_ref[...], wau_ref[...],
                             preferred_element_type=jnp.float32) + bau_ref[...], 0.0)
    y = jnp.dot(hl, w2f_ref[...], preferred_element_type=jnp.float32)
    acc = jnp.dot(hl, bl_ref[...], preferred_element_type=jnp.float32)
    ang = ang_ref[...]
    el = g2_ref[:, 0:1]
    for k in range(K):
        acc += y[:, k * H:(k + 1) * H] * _rbf_col(ang, k)
    for k in range(K):
        acc += y[:, (K + k) * H:(K + k + 1) * H] * _rbf_col(el, k)
    rows = i * BLK + lax.broadcasted_iota(jnp.int32, (BLK, 1), 0)
    o_ref[...] = jnp.where(rows < E2, acc, 0.0)


def _line_msg(ang_p, g2, bm2, wau, bau, w2f, bline):
    return pl.pallas_call(
        _kb_body,
        grid=(E2P // BLK,),
        in_specs=[
            pl.BlockSpec((BLK, 1), lambda i: (i, 0)),
            pl.BlockSpec((BLK, 16), lambda i: (i, 0)),
            pl.BlockSpec((BLK, H), lambda i: (i, 0)),
            pl.BlockSpec((H, H), lambda i: (0, 0)),
            pl.BlockSpec((1, H), lambda i: (0, 0)),
            pl.BlockSpec((H, 16 * H), lambda i: (0, 0)),
            pl.BlockSpec((H, H), lambda i: (0, 0)),
        ],
        out_specs=pl.BlockSpec((BLK, H), lambda i: (i, 0)),
        out_shape=jax.ShapeDtypeStruct((E2P, H), jnp.float32),
        interpret=_INTERP,
    )(ang_p, g2, bm2, wau, bau, w2f, bline)


def _kc_body(ea_ref, el_ref, bm_ref, ap_ref, wbu_ref, bbu_ref, wbf_ref, bb_ref, o_ref):
    i = pl.program_id(0)
    hb = jnp.maximum(jnp.dot(bm_ref[...], wbu_ref[...],
                             preferred_element_type=jnp.float32) + bbu_ref[...], 0.0)
    hb = hb + ap_ref[0] + ap_ref[1]
    y = jnp.dot(hb, wbf_ref[...], preferred_element_type=jnp.float32)
    acc = jnp.dot(hb, bb_ref[...], preferred_element_type=jnp.float32)
    ea = ea_ref[...]
    el = el_ref[...]
    for k in range(D_EATTR):
        acc += y[:, k * H:(k + 1) * H] * ea[:, k:k + 1]
    for k in range(K):
        acc += y[:, (D_EATTR + k) * H:(D_EATTR + k + 1) * H] * _rbf_col(el, k)
    rows = i * BLK + lax.broadcasted_iota(jnp.int32, (BLK, 1), 0)
    o_ref[...] = jnp.where(rows < E, acc, 0.0)


def _bond_msg(ea_p, el_p, bm, ap, wbu, bbu, wbf, bbond):
    return pl.pallas_call(
        _kc_body,
        grid=(EP // BLK,),
        in_specs=[
            pl.BlockSpec((BLK, D_EATTR), lambda i: (i, 0)),
            pl.BlockSpec((BLK, 1), lambda i: (i, 0)),
            pl.BlockSpec((BLK, H), lambda i: (i, 0)),
            pl.BlockSpec((2, BLK, H), lambda i: (0, i, 0)),
            pl.BlockSpec((H, H), lambda i: (0, 0)),
            pl.BlockSpec((1, H), lambda i: (0, 0)),
            pl.BlockSpec((H, 16 * H), lambda i: (0, 0)),
            pl.BlockSpec((H, H), lambda i: (0, 0)),
        ],
        out_specs=pl.BlockSpec((BLK, H), lambda i: (i, 0)),
        out_shape=jax.ShapeDtypeStruct((EP, H), jnp.float32),
        interpret=_INTERP,
    )(ea_p, el_p, bm, ap, wbu, bbu, wbf, bbond)


def _kd_body(np_ref, h_ref, gb_ref, wih_ref, whh_ref, bih_ref, bhh_ref, o_ref):
    x = jnp.maximum(np_ref[0] + np_ref[1] + gb_ref[...], 0.0)
    h = h_ref[...]
    gi = jnp.dot(x, wih_ref[...].T, preferred_element_type=jnp.float32) + bih_ref[...]
    gh = jnp.dot(h, whh_ref[...].T, preferred_element_type=jnp.float32) + bhh_ref[...]
    r = jax.nn.sigmoid(gi[:, :H] + gh[:, :H])
    z = jax.nn.sigmoid(gi[:, H:2 * H] + gh[:, H:2 * H])
    n = jnp.tanh(gi[:, 2 * H:] + r * gh[:, 2 * H:])
    o_ref[...] = (1.0 - z) * n + z * h


def _gru_step(npart, h_gru, gnn_bias, wih, whh, bih, bhh):
    return pl.pallas_call(
        _kd_body,
        grid=(NP // BLK,),
        in_specs=[
            pl.BlockSpec((2, BLK, H), lambda i: (0, i, 0)),
            pl.BlockSpec((BLK, H), lambda i: (i, 0)),
            pl.BlockSpec((1, H), lambda i: (0, 0)),
            pl.BlockSpec((3 * H, H), lambda i: (0, 0)),
            pl.BlockSpec((3 * H, H), lambda i: (0, 0)),
            pl.BlockSpec((1, 3 * H), lambda i: (0, 0)),
            pl.BlockSpec((1, 3 * H), lambda i: (0, 0)),
        ],
        out_specs=pl.BlockSpec((BLK, H), lambda i: (i, 0)),
        out_shape=jax.ShapeDtypeStruct((NP, H), jnp.float32),
        interpret=_INTERP,
    )(npart, h_gru, gnn_bias, wih, whh, bih, bhh)


def _lstm(x, h, c, wih, whh, bih, bhh):
    d = h.shape[-1]
    g = (jnp.dot(x, wih.T, preferred_element_type=jnp.float32) + bih
         + jnp.dot(h, whh.T, preferred_element_type=jnp.float32) + bhh)
    i = jax.nn.sigmoid(g[:, :d])
    f = jax.nn.sigmoid(g[:, d:2 * d])
    gg = jnp.tanh(g[:, 2 * d:3 * d])
    o = jax.nn.sigmoid(g[:, 3 * d:])
    c2 = f * c + i * gg
    return o * jnp.tanh(c2), c2


def _kz_body(x_ref, nf_ref, wih0_ref, whh0_ref, bih0_ref, bhh0_ref,
             wih1_ref, whh1_ref, bih1_ref, bhh1_ref, wsp_ref, bsp_ref, pa_ref,
             o_ref):
    na = jnp.concatenate([x_ref[...], nf_ref[...]], axis=1)
    rows = lax.broadcasted_iota(jnp.int32, (NP, 1), 0)
    valid = rows < N
    d = 2 * H
    q_star = jnp.zeros((1, 2 * d), jnp.float32)
    h0 = jnp.zeros((1, d), jnp.float32)
    c0 = jnp.zeros((1, d), jnp.float32)
    h1 = jnp.zeros((1, d), jnp.float32)
    c1 = jnp.zeros((1, d), jnp.float32)
    for _ in range(POOL_ITERS):
        h0, c0 = _lstm(q_star, h0, c0, wih0_ref[...], whh0_ref[...],
                       bih0_ref[...], bhh0_ref[...])
        h1, c1 = _lstm(h0, h1, c1, wih1_ref[...], whh1_ref[...],
                       bih1_ref[...], bhh1_ref[...])
        q = h1
        e = jnp.sum(na * q, axis=-1, keepdims=True)
        e = jnp.where(valid, e, -1e30)
        m = jnp.max(e, axis=0, keepdims=True)
        p = jnp.where(valid, jnp.exp(e - m), 0.0)
        alpha = p / jnp.sum(p, axis=0, keepdims=True)
        readout = jnp.sum(na * alpha, axis=0, keepdims=True)
        q_star = jnp.concatenate([q, readout], axis=-1)
    y = jnp.dot(q_star, wsp_ref[...], preferred_element_type=jnp.float32) + bsp_ref[...]
    o_ref[...] = jnp.where(y >= 0.0, y, pa_ref[...] * y)


def _set2set(x, nf, wih0, whh0, bih0, bhh0, wih1, whh1, bih1, bhh1, wsp, bsp, pa):
    full = lambda s: pl.BlockSpec(s, lambda: tuple(0 for _ in s))
    return pl.pallas_call(
        _kz_body,
        in_specs=[
            full((NP, H)), full((NP, H)),
            full((4 * 2 * H, 4 * H)), full((4 * 2 * H, 2 * H)),
            full((1, 4 * 2 * H)), full((1, 4 * 2 * H)),
            full((4 * 2 * H, 2 * H)), full((4 * 2 * H, 2 * H)),
            full((1, 4 * 2 * H)), full((1, 4 * 2 * H)),
            full((4 * H, D_HID)), full((1, D_HID)), full((1, 1)),
        ],
        out_specs=full((1, D_HID)),
        out_shape=jax.ShapeDtypeStruct((1, D_HID), jnp.float32),
        interpret=_INTERP,
    )(x, nf, wih0, whh0, bih0, bhh0, wih1, whh1, bih1, bhh1, wsp, bsp, pa)


_GRP = 13


def _sc_gather(table, idx3):
    _, C, _ = idx3.shape
    D = table.shape[1]
    groups = [(g, min(_GRP, C - g)) for g in range(0, C, _GRP)]
    mesh = plsc.VectorSubcoreMesh(core_axis_name="c", subcore_axis_name="s")

    @functools.partial(
        pl.kernel,
        out_type=jax.ShapeDtypeStruct((NW, C, CH, D), jnp.float32),
        mesh=mesh,
        compiler_params=pltpu.CompilerParams(use_tc_tiling_on_sc=False),
        scratch_types=[
            pltpu.VMEM((C, CH), jnp.int32),
            pltpu.VMEM((_GRP, CH, D), jnp.float32),
            pltpu.SemaphoreType.DMA,
        ],
    )
    def k(table_hbm, idx_hbm, out_hbm, idx_v, buf_v, sem):
        cid = lax.axis_index("c")
        sid = lax.axis_index("s")
        wid = sid * 2 + cid
        pltpu.sync_copy(idx_hbm.at[wid], idx_v)
        for g0, gsz in groups:
            cps = [pltpu.async_copy(table_hbm.at[idx_v.at[g0 + j]], buf_v.at[j], sem)
                   for j in range(gsz)]
            for cp in cps:
                cp.wait()
            pltpu.sync_copy(buf_v.at[pl.ds(0, gsz)], out_hbm.at[wid, pl.ds(g0, gsz)])

    return k(table, idx3)


def _gather_rows(table, idx_pad):
    B = idx_pad.shape[0]
    C = B // (NW * CH)
    out = _sc_gather(table, idx_pad.reshape(NW, C, CH))
    return out.reshape(B, table.shape[1])


def _scatter_add(payload, idx_pad, nseg):
    seg = jax.ops.segment_sum(payload, idx_pad, num_segments=nseg)
    return jnp.stack([seg, jnp.zeros_like(seg)])


def kernel(node_attr, edge_index, edge_attr, edge_length, ee_index, ee_angle,
           W_proj, b_proj, W_bond, b_bond, W_edgefn, b_edgefn, W_bu, b_bu,
           W_au, b_au, gnn_bias, gru_Wih, gru_Whh, gru_bih, gru_bhh,
           s2s_Wih0, s2s_Whh0, s2s_bih0, s2s_bhh0,
           s2s_Wih1, s2s_Whh1, s2s_bih1, s2s_bhh1, W_sp, b_sp, prelu_a):
    f32 = jnp.float32
    src = edge_index[0].astype(jnp.int32)
    dst = edge_index[1].astype(jnp.int32)
    ee_src = ee_index[0].astype(jnp.int32)
    ee_dst = ee_index[1].astype(jnp.int32)

    na_pad = jnp.pad(node_attr, ((0, NP - N), (0, 112 - D_NODE)))
    wp_pad = jnp.pad(W_proj, ((0, 112 - D_NODE), (0, 0)))
    w2f = W_edgefn.reshape(16, H, H).transpose(1, 0, 2).reshape(H, 16 * H)
    bline = b_edgefn.reshape(H, H)
    wbf = W_bond.reshape(16, H, H).transpose(1, 0, 2).reshape(H, 16 * H)
    bbond = b_bond.reshape(H, H)

    ee_src_p = jnp.pad(ee_src, (0, E2P - E2))
    ee_dst_p = jnp.pad(ee_dst, (0, E2P - E2))
    dst_p = jnp.pad(dst, (0, EP - E))
    src_p = jnp.pad(src, (0, EP - E))
    ang_p = jnp.pad(ee_angle, (0, E2P - E2))[:, None]
    ea_p = jnp.pad(edge_attr, ((0, EP - E), (0, 0)))
    el_p = jnp.pad(edge_length, (0, EP - E))[:, None]

    src_bits = lax.bitcast_convert_type(src, f32)
    table16 = jnp.zeros((E, 16), f32)
    table16 = table16.at[:, 0].set(edge_length)
    table16 = table16.at[:, 1].set(src_bits)

    g2 = _gather_rows(table16, ee_src_p)
    src2_p = lax.bitcast_convert_type(g2[:, 1], jnp.int32)
    idxg = jnp.concatenate([src_p, src2_p])

    nf = _node_proj(na_pad, wp_pad, b_proj[None])

    x = nf
    h_gru = nf
    for _ in range(STEPS):
        g = _gather_rows(x, idxg)
        bm = g[:EP]
        bm2 = g[EP:]
        out_line = _line_msg(ang_p, g2, bm2, W_au, b_au[None], w2f, bline)
        ap = _scatter_add(out_line, ee_dst_p, EP)
        m = _bond_msg(ea_p, el_p, bm, ap, W_bu, b_bu[None], wbf, bbond)
        npart = _scatter_add(m, dst_p, NP)
        x = _gru_step(npart, h_gru, gnn_bias[None], gru_Wih, gru_Whh,
                      gru_bih[None], gru_bhh[None])
        h_gru = x

    return _set2set(x, nf, s2s_Wih0, s2s_Whh0, s2s_bih0[None], s2s_bhh0[None],
                    s2s_Wih1, s2s_Whh1, s2s_bih1[None], s2s_bhh1[None],
                    W_sp, b_sp[None], prelu_a.reshape(1, 1))


# speedup vs baseline: 12.6639x; 1.1444x over previous
import functools

import jax
import jax.numpy as jnp
from jax import lax
from jax.experimental import pallas as pl
from jax.experimental.pallas import tpu as pltpu
from jax.experimental.pallas import tpu_sc as plsc

N = 20000
E = 50000
E2 = 100000
D_NODE = 110
D_EATTR = 8
K = 8
H = 32
D_HID = 4096
STEPS = 2
POOL_ITERS = 3

CUTOFF = 5.0
GAMMA = 10.0
CENTERS = [CUTOFF * i / (K - 1) for i in range(K)]

NW = 32
CH = 128
C_E = 13
C_E2 = 25
C_G = C_E + C_E2
EP = NW * C_E * CH
E2P = NW * C_E2 * CH
BG = NW * C_G * CH
NP = 20480
BLK = 2048

_INTERP = False


def _rbf_col(d, k):
    return jnp.exp(-GAMMA * (d - CENTERS[k]) ** 2)


def _k0_body(na_ref, wp_ref, bp_ref, o_ref):
    i = pl.program_id(0)
    x = jnp.maximum(jnp.dot(na_ref[...], wp_ref[...],
                            preferred_element_type=jnp.float32) + bp_ref[...], 0.0)
    rows = i * BLK + lax.broadcasted_iota(jnp.int32, (BLK, 1), 0)
    o_ref[...] = jnp.where(rows < N, x, 0.0)


def _node_proj(na_pad, wp_pad, bp):
    return pl.pallas_call(
        _k0_body,
        grid=(NP // BLK,),
        in_specs=[
            pl.BlockSpec((BLK, 112), lambda i: (i, 0)),
            pl.BlockSpec((112, H), lambda i: (0, 0)),
            pl.BlockSpec((1, H), lambda i: (0, 0)),
        ],
        out_specs=pl.BlockSpec((BLK, H), lambda i: (i, 0)),
        out_shape=jax.ShapeDtypeStruct((NP, H), jnp.float32),
        interpret=_INTERP,
    )(na_pad, wp_pad, bp)


def _kb_body(ang_ref, g2_ref, bm2_ref, wau_ref, bau_ref, w2f_ref, bl_ref, o_ref):
    i = pl.program_id(0)
    hl = jnp.maximum(jnp.dot(bm2_ref[...], wau_ref[...],
                             preferred_element_type=jnp.float32) + bau_ref[...], 0.0)
    y = jnp.dot(hl, w2f_ref[...], preferred_element_type=jnp.float32)
    acc = jnp.dot(hl, bl_ref[...], preferred_element_type=jnp.float32)
    ang = ang_ref[...]
    el = g2_ref[:, 0:1]
    for k in range(K):
        acc += y[:, k * H:(k + 1) * H] * _rbf_col(ang, k)
    for k in range(K):
        acc += y[:, (K + k) * H:(K + k + 1) * H] * _rbf_col(el, k)
    rows = i * BLK + lax.broadcasted_iota(jnp.int32, (BLK, 1), 0)
    o_ref[...] = jnp.where(rows < E2, acc, 0.0)


def _line_msg(ang_p, g2, bm2, wau, bau, w2f, bline):
    return pl.pallas_call(
        _kb_body,
        grid=(E2P // BLK,),
        in_specs=[
            pl.BlockSpec((BLK, 1), lambda i: (i, 0)),
            pl.BlockSpec((BLK, 16), lambda i: (i, 0)),
            pl.BlockSpec((BLK, H), lambda i: (i, 0)),
            pl.BlockSpec((H, H), lambda i: (0, 0)),
            pl.BlockSpec((1, H), lambda i: (0, 0)),
            pl.BlockSpec((H, 16 * H), lambda i: (0, 0)),
            pl.BlockSpec((H, H), lambda i: (0, 0)),
        ],
        out_specs=pl.BlockSpec((BLK, H), lambda i: (i, 0)),
        out_shape=jax.ShapeDtypeStruct((E2P, H), jnp.float32),
        interpret=_INTERP,
    )(ang_p, g2, bm2, wau, bau, w2f, bline)


def _kc_body(ea_ref, el_ref, bm_ref, ap_ref, wbu_ref, bbu_ref, wbf_ref, bb_ref, o_ref):
    i = pl.program_id(0)
    hb = jnp.maximum(jnp.dot(bm_ref[...], wbu_ref[...],
                             preferred_element_type=jnp.float32) + bbu_ref[...], 0.0)
    hb = hb + ap_ref[0] + ap_ref[1]
    y = jnp.dot(hb, wbf_ref[...], preferred_element_type=jnp.float32)
    acc = jnp.dot(hb, bb_ref[...], preferred_element_type=jnp.float32)
    ea = ea_ref[...]
    el = el_ref[...]
    for k in range(D_EATTR):
        acc += y[:, k * H:(k + 1) * H] * ea[:, k:k + 1]
    for k in range(K):
        acc += y[:, (D_EATTR + k) * H:(D_EATTR + k + 1) * H] * _rbf_col(el, k)
    rows = i * BLK + lax.broadcasted_iota(jnp.int32, (BLK, 1), 0)
    o_ref[...] = jnp.where(rows < E, acc, 0.0)


def _bond_msg(ea_p, el_p, bm, ap, wbu, bbu, wbf, bbond):
    return pl.pallas_call(
        _kc_body,
        grid=(EP // BLK,),
        in_specs=[
            pl.BlockSpec((BLK, D_EATTR), lambda i: (i, 0)),
            pl.BlockSpec((BLK, 1), lambda i: (i, 0)),
            pl.BlockSpec((BLK, H), lambda i: (i, 0)),
            pl.BlockSpec((2, BLK, H), lambda i: (0, i, 0)),
            pl.BlockSpec((H, H), lambda i: (0, 0)),
            pl.BlockSpec((1, H), lambda i: (0, 0)),
            pl.BlockSpec((H, 16 * H), lambda i: (0, 0)),
            pl.BlockSpec((H, H), lambda i: (0, 0)),
        ],
        out_specs=pl.BlockSpec((BLK, H), lambda i: (i, 0)),
        out_shape=jax.ShapeDtypeStruct((EP, H), jnp.float32),
        interpret=_INTERP,
    )(ea_p, el_p, bm, ap, wbu, bbu, wbf, bbond)


def _kd_body(np_ref, h_ref, gb_ref, wih_ref, whh_ref, bih_ref, bhh_ref, o_ref):
    x = jnp.maximum(np_ref[0] + np_ref[1] + gb_ref[...], 0.0)
    h = h_ref[...]
    gi = jnp.dot(x, wih_ref[...].T, preferred_element_type=jnp.float32) + bih_ref[...]
    gh = jnp.dot(h, whh_ref[...].T, preferred_element_type=jnp.float32) + bhh_ref[...]
    r = jax.nn.sigmoid(gi[:, :H] + gh[:, :H])
    z = jax.nn.sigmoid(gi[:, H:2 * H] + gh[:, H:2 * H])
    n = jnp.tanh(gi[:, 2 * H:] + r * gh[:, 2 * H:])
    o_ref[...] = (1.0 - z) * n + z * h


def _gru_step(npart, h_gru, gnn_bias, wih, whh, bih, bhh):
    return pl.pallas_call(
        _kd_body,
        grid=(NP // BLK,),
        in_specs=[
            pl.BlockSpec((2, BLK, H), lambda i: (0, i, 0)),
            pl.BlockSpec((BLK, H), lambda i: (i, 0)),
            pl.BlockSpec((1, H), lambda i: (0, 0)),
            pl.BlockSpec((3 * H, H), lambda i: (0, 0)),
            pl.BlockSpec((3 * H, H), lambda i: (0, 0)),
            pl.BlockSpec((1, 3 * H), lambda i: (0, 0)),
            pl.BlockSpec((1, 3 * H), lambda i: (0, 0)),
        ],
        out_specs=pl.BlockSpec((BLK, H), lambda i: (i, 0)),
        out_shape=jax.ShapeDtypeStruct((NP, H), jnp.float32),
        interpret=_INTERP,
    )(npart, h_gru, gnn_bias, wih, whh, bih, bhh)


def _lstm(x, h, c, wih, whh, bih, bhh):
    d = h.shape[-1]
    g = (jnp.dot(x, wih.T, preferred_element_type=jnp.float32) + bih
         + jnp.dot(h, whh.T, preferred_element_type=jnp.float32) + bhh)
    i = jax.nn.sigmoid(g[:, :d])
    f = jax.nn.sigmoid(g[:, d:2 * d])
    gg = jnp.tanh(g[:, 2 * d:3 * d])
    o = jax.nn.sigmoid(g[:, 3 * d:])
    c2 = f * c + i * gg
    return o * jnp.tanh(c2), c2


def _kz_body(x_ref, nf_ref, wih0_ref, whh0_ref, bih0_ref, bhh0_ref,
             wih1_ref, whh1_ref, bih1_ref, bhh1_ref, wsp_ref, bsp_ref, pa_ref,
             o_ref):
    na = jnp.concatenate([x_ref[...], nf_ref[...]], axis=1)
    rows = lax.broadcasted_iota(jnp.int32, (NP, 1), 0)
    valid = rows < N
    d = 2 * H
    q_star = jnp.zeros((1, 2 * d), jnp.float32)
    h0 = jnp.zeros((1, d), jnp.float32)
    c0 = jnp.zeros((1, d), jnp.float32)
    h1 = jnp.zeros((1, d), jnp.float32)
    c1 = jnp.zeros((1, d), jnp.float32)
    for _ in range(POOL_ITERS):
        h0, c0 = _lstm(q_star, h0, c0, wih0_ref[...], whh0_ref[...],
                       bih0_ref[...], bhh0_ref[...])
        h1, c1 = _lstm(h0, h1, c1, wih1_ref[...], whh1_ref[...],
                       bih1_ref[...], bhh1_ref[...])
        q = h1
        e = jnp.sum(na * q, axis=-1, keepdims=True)
        e = jnp.where(valid, e, -1e30)
        m = jnp.max(e, axis=0, keepdims=True)
        p = jnp.where(valid, jnp.exp(e - m), 0.0)
        alpha = p / jnp.sum(p, axis=0, keepdims=True)
        readout = jnp.sum(na * alpha, axis=0, keepdims=True)
        q_star = jnp.concatenate([q, readout], axis=-1)
    y = jnp.dot(q_star, wsp_ref[...], preferred_element_type=jnp.float32) + bsp_ref[...]
    o_ref[...] = jnp.where(y >= 0.0, y, pa_ref[...] * y)


def _set2set(x, nf, wih0, whh0, bih0, bhh0, wih1, whh1, bih1, bhh1, wsp, bsp, pa):
    full = lambda s: pl.BlockSpec(s, lambda: tuple(0 for _ in s))
    return pl.pallas_call(
        _kz_body,
        in_specs=[
            full((NP, H)), full((NP, H)),
            full((4 * 2 * H, 4 * H)), full((4 * 2 * H, 2 * H)),
            full((1, 4 * 2 * H)), full((1, 4 * 2 * H)),
            full((4 * 2 * H, 2 * H)), full((4 * 2 * H, 2 * H)),
            full((1, 4 * 2 * H)), full((1, 4 * 2 * H)),
            full((4 * H, D_HID)), full((1, D_HID)), full((1, 1)),
        ],
        out_specs=full((1, D_HID)),
        out_shape=jax.ShapeDtypeStruct((1, D_HID), jnp.float32),
        interpret=_INTERP,
    )(x, nf, wih0, whh0, bih0, bhh0, wih1, whh1, bih1, bhh1, wsp, bsp, pa)


_GRP = 13


def _sc_gather(table, idx3):
    _, C, _ = idx3.shape
    D = table.shape[1]
    groups = [(g, min(_GRP, C - g)) for g in range(0, C, _GRP)]
    mesh = plsc.VectorSubcoreMesh(core_axis_name="c", subcore_axis_name="s")

    @functools.partial(
        pl.kernel,
        out_type=jax.ShapeDtypeStruct((NW, C, CH, D), jnp.float32),
        mesh=mesh,
        compiler_params=pltpu.CompilerParams(use_tc_tiling_on_sc=False),
        scratch_types=[
            pltpu.VMEM((C, CH), jnp.int32),
            pltpu.VMEM((_GRP, CH, D), jnp.float32),
            pltpu.SemaphoreType.DMA,
        ],
    )
    def k(table_hbm, idx_hbm, out_hbm, idx_v, buf_v, sem):
        cid = lax.axis_index("c")
        sid = lax.axis_index("s")
        wid = sid * 2 + cid
        pltpu.sync_copy(idx_hbm.at[wid], idx_v)
        for g0, gsz in groups:
            cps = [pltpu.async_copy(table_hbm.at[idx_v.at[g0 + j]], buf_v.at[j], sem)
                   for j in range(gsz)]
            for cp in cps:
                cp.wait()
            pltpu.sync_copy(buf_v.at[pl.ds(0, gsz)], out_hbm.at[wid, pl.ds(g0, gsz)])

    return k(table, idx3)


def _gather_rows(table, idx_pad):
    B = idx_pad.shape[0]
    C = B // (NW * CH)
    out = _sc_gather(table, idx_pad.reshape(NW, C, CH))
    return out.reshape(B, table.shape[1])


def _sc_scatter_add(payload4, idx3, zeros, s_acc, s_out):
    _, C, _, D = payload4.shape
    rpt_i = s_acc // 16
    rpt_o = s_out // 16
    groups = [(g, min(_GRP, C - g)) for g in range(0, C, _GRP)]
    mesh = plsc.VectorSubcoreMesh(core_axis_name="c", subcore_axis_name="s")

    @functools.partial(
        pl.kernel,
        out_type=jax.ShapeDtypeStruct((2, s_out, D), jnp.float32),
        mesh=mesh,
        compiler_params=pltpu.CompilerParams(use_tc_tiling_on_sc=False),
        scratch_types=[
            pltpu.VMEM((C, CH), jnp.int32),
            pltpu.VMEM((_GRP, CH, D), jnp.float32),
            pltpu.VMEM_SHARED((s_acc, D), jnp.float32),
            pltpu.SemaphoreType.DMA,
        ],
    )
    def k(pay_hbm, idx_hbm, z_hbm, out_hbm, idx_v, buf_v, acc_sh, sem):
        cid = lax.axis_index("c")
        sid = lax.axis_index("s")
        wid = sid * 2 + cid
        pltpu.sync_copy(z_hbm.at[pl.ds(sid * rpt_i, rpt_i)],
                        acc_sh.at[pl.ds(sid * rpt_i, rpt_i)])
        pltpu.sync_copy(idx_hbm.at[wid], idx_v)
        plsc.subcore_barrier()
        for g0, gsz in groups:
            pltpu.sync_copy(pay_hbm.at[wid, pl.ds(g0, gsz)], buf_v.at[pl.ds(0, gsz)])
            cps = [pltpu.async_copy(buf_v.at[j], acc_sh.at[idx_v.at[g0 + j]],
                                    sem, add=True)
                   for j in range(gsz)]
            for cp in cps:
                cp.wait()
        plsc.subcore_barrier()
        pltpu.sync_copy(acc_sh.at[pl.ds(sid * rpt_o, rpt_o)],
                        out_hbm.at[cid, pl.ds(sid * rpt_o, rpt_o)])

    return k(payload4, idx3, zeros)


def _scatter_add(payload, idx_pad, nseg):
    B, D = payload.shape
    C = B // (NW * CH)
    pay4 = payload.reshape(NW, C, CH, D)
    if nseg * D <= 1024 * 1024:
        zeros = jnp.zeros((nseg, D), jnp.float32)
        return _sc_scatter_add(pay4, idx_pad.reshape(NW, C, CH), zeros,
                               nseg, nseg)
    hp = nseg // 2
    s_acc = hp + 128
    zeros = jnp.zeros((s_acc, D), jnp.float32)
    idx_a = jnp.where(idx_pad < hp, idx_pad, hp).reshape(NW, C, CH)
    idx_b = jnp.where(idx_pad >= hp, idx_pad - hp, hp).reshape(NW, C, CH)
    out_a = _sc_scatter_add(pay4, idx_a, zeros, s_acc, hp)
    out_b = _sc_scatter_add(pay4, idx_b, zeros, s_acc, hp)
    return jnp.stack([out_a, out_b], axis=1).reshape(2, nseg, D)


def kernel(node_attr, edge_index, edge_attr, edge_length, ee_index, ee_angle,
           W_proj, b_proj, W_bond, b_bond, W_edgefn, b_edgefn, W_bu, b_bu,
           W_au, b_au, gnn_bias, gru_Wih, gru_Whh, gru_bih, gru_bhh,
           s2s_Wih0, s2s_Whh0, s2s_bih0, s2s_bhh0,
           s2s_Wih1, s2s_Whh1, s2s_bih1, s2s_bhh1, W_sp, b_sp, prelu_a):
    f32 = jnp.float32
    src = edge_index[0].astype(jnp.int32)
    dst = edge_index[1].astype(jnp.int32)
    ee_src = ee_index[0].astype(jnp.int32)
    ee_dst = ee_index[1].astype(jnp.int32)

    na_pad = jnp.pad(node_attr, ((0, NP - N), (0, 112 - D_NODE)))
    wp_pad = jnp.pad(W_proj, ((0, 112 - D_NODE), (0, 0)))
    w2f = W_edgefn.reshape(16, H, H).transpose(1, 0, 2).reshape(H, 16 * H)
    bline = b_edgefn.reshape(H, H)
    wbf = W_bond.reshape(16, H, H).transpose(1, 0, 2).reshape(H, 16 * H)
    bbond = b_bond.reshape(H, H)

    ee_src_p = jnp.pad(ee_src, (0, E2P - E2))
    ee_dst_p = jnp.pad(ee_dst, (0, E2P - E2))
    dst_p = jnp.pad(dst, (0, EP - E))
    src_p = jnp.pad(src, (0, EP - E))
    ang_p = jnp.pad(ee_angle, (0, E2P - E2))[:, None]
    ea_p = jnp.pad(edge_attr, ((0, EP - E), (0, 0)))
    el_p = jnp.pad(edge_length, (0, EP - E))[:, None]

    src_bits = lax.bitcast_convert_type(src, f32)
    table16 = jnp.zeros((E, 16), f32)
    table16 = table16.at[:, 0].set(edge_length)
    table16 = table16.at[:, 1].set(src_bits)

    g2 = _gather_rows(table16, ee_src_p)
    src2_p = lax.bitcast_convert_type(g2[:, 1], jnp.int32)
    idxg = jnp.concatenate([src_p, src2_p])

    nf = _node_proj(na_pad, wp_pad, b_proj[None])

    x = nf
    h_gru = nf
    for _ in range(STEPS):
        g = _gather_rows(x, idxg)
        bm = g[:EP]
        bm2 = g[EP:]
        out_line = _line_msg(ang_p, g2, bm2, W_au, b_au[None], w2f, bline)
        ap = _scatter_add(out_line, ee_dst_p, EP)
        m = _bond_msg(ea_p, el_p, bm, ap, W_bu, b_bu[None], wbf, bbond)
        npart = _scatter_add(m, dst_p, NP)
        x = _gru_step(npart, h_gru, gnn_bias[None], gru_Wih, gru_Whh,
                      gru_bih[None], gru_bhh[None])
        h_gru = x

    return _set2set(x, nf, s2s_Wih0, s2s_Whh0, s2s_bih0[None], s2s_bhh0[None],
                    s2s_Wih1, s2s_Whh1, s2s_bih1[None], s2s_bhh1[None],
                    W_sp, b_sp[None], prelu_a.reshape(1, 1))


# speedup vs baseline: 20.7361x; 1.6374x over previous
import functools

import jax
import jax.numpy as jnp
from jax import lax
from jax.experimental import pallas as pl
from jax.experimental.pallas import tpu as pltpu
from jax.experimental.pallas import tpu_sc as plsc

N = 20000
E = 50000
E2 = 100000
D_NODE = 110
D_EATTR = 8
K = 8
H = 32
D_HID = 4096
STEPS = 2
POOL_ITERS = 3

CUTOFF = 5.0
GAMMA = 10.0
CENTERS = [CUTOFF * i / (K - 1) for i in range(K)]

NW = 32
CH = 128
C_E = 13
C_E2 = 25
C_G = C_E + C_E2
EP = NW * C_E * CH
E2P = NW * C_E2 * CH
BG = NW * C_G * CH
NP = 20480
BLK = 2048

_INTERP = False


def _rbf_col(d, k):
    return jnp.exp(-GAMMA * (d - CENTERS[k]) ** 2)


def _k0_body(na_ref, wp_ref, bp_ref, o_ref):
    i = pl.program_id(0)
    x = jnp.maximum(jnp.dot(na_ref[...], wp_ref[...],
                            preferred_element_type=jnp.float32) + bp_ref[...], 0.0)
    rows = i * BLK + lax.broadcasted_iota(jnp.int32, (BLK, 1), 0)
    o_ref[...] = jnp.where(rows < N, x, 0.0)


def _node_proj(na_pad, wp_pad, bp):
    return pl.pallas_call(
        _k0_body,
        grid=(NP // BLK,),
        in_specs=[
            pl.BlockSpec((BLK, 112), lambda i: (i, 0)),
            pl.BlockSpec((112, H), lambda i: (0, 0)),
            pl.BlockSpec((1, H), lambda i: (0, 0)),
        ],
        out_specs=pl.BlockSpec((BLK, H), lambda i: (i, 0)),
        out_shape=jax.ShapeDtypeStruct((NP, H), jnp.float32),
        interpret=_INTERP,
    )(na_pad, wp_pad, bp)


def _dotT(a, b):
    return lax.dot_general(a, b, (((0,), (1,)), ((), ())),
                           preferred_element_type=jnp.float32)


def _dot00(a, b):
    return lax.dot_general(a, b, (((0,), (0,)), ((), ())),
                           preferred_element_type=jnp.float32)


def _kb_body(angT_ref, elT_ref, bm2_ref, wau_ref, bauT_ref, w2f_ref, bl_ref, o_ref):
    i = pl.program_id(0)
    hlT = jnp.maximum(_dotT(wau_ref[...], bm2_ref[...]) + bauT_ref[...], 0.0)
    yT = _dot00(w2f_ref[...], hlT)
    accT = _dot00(bl_ref[...], hlT)
    angT = angT_ref[...]
    elT = elT_ref[...]
    for k in range(K):
        accT += yT[k * H:(k + 1) * H] * _rbf_col(angT, k)
    for k in range(K):
        accT += yT[(K + k) * H:(K + k + 1) * H] * _rbf_col(elT, k)
    rows = i * BLK + lax.broadcasted_iota(jnp.int32, (BLK, 1), 0)
    o_ref[...] = jnp.where(rows < E2, accT.T, 0.0)


def _line_msg(angT, el2T, bm2, wau, bauT, w2f, bline):
    return pl.pallas_call(
        _kb_body,
        grid=(E2P // BLK,),
        in_specs=[
            pl.BlockSpec((1, BLK), lambda i: (0, i)),
            pl.BlockSpec((1, BLK), lambda i: (0, i)),
            pl.BlockSpec((BLK, H), lambda i: (i, 0)),
            pl.BlockSpec((H, H), lambda i: (0, 0)),
            pl.BlockSpec((H, 1), lambda i: (0, 0)),
            pl.BlockSpec((H, 16 * H), lambda i: (0, 0)),
            pl.BlockSpec((H, H), lambda i: (0, 0)),
        ],
        out_specs=pl.BlockSpec((BLK, H), lambda i: (i, 0)),
        out_shape=jax.ShapeDtypeStruct((E2P, H), jnp.float32),
        interpret=_INTERP,
    )(angT, el2T, bm2, wau, bauT, w2f, bline)


def _kc_body(eaT_ref, elT_ref, bm_ref, ap_ref, wbu_ref, bbuT_ref, wbf_ref, bb_ref, o_ref):
    i = pl.program_id(0)
    hbT = jnp.maximum(_dotT(wbu_ref[...], bm_ref[...]) + bbuT_ref[...], 0.0)
    hbT = hbT + (ap_ref[0] + ap_ref[1]).T
    yT = _dot00(wbf_ref[...], hbT)
    accT = _dot00(bb_ref[...], hbT)
    eaT = eaT_ref[...]
    elT = elT_ref[...]
    for k in range(D_EATTR):
        accT += yT[k * H:(k + 1) * H] * eaT[k:k + 1]
    for k in range(K):
        accT += yT[(D_EATTR + k) * H:(D_EATTR + k + 1) * H] * _rbf_col(elT, k)
    rows = i * BLK + lax.broadcasted_iota(jnp.int32, (BLK, 1), 0)
    o_ref[...] = jnp.where(rows < E, accT.T, 0.0)


def _bond_msg(eaT, elT, bm, ap, wbu, bbuT, wbf, bbond):
    return pl.pallas_call(
        _kc_body,
        grid=(EP // BLK,),
        in_specs=[
            pl.BlockSpec((D_EATTR, BLK), lambda i: (0, i)),
            pl.BlockSpec((1, BLK), lambda i: (0, i)),
            pl.BlockSpec((BLK, H), lambda i: (i, 0)),
            pl.BlockSpec((2, BLK, H), lambda i: (0, i, 0)),
            pl.BlockSpec((H, H), lambda i: (0, 0)),
            pl.BlockSpec((H, 1), lambda i: (0, 0)),
            pl.BlockSpec((H, 16 * H), lambda i: (0, 0)),
            pl.BlockSpec((H, H), lambda i: (0, 0)),
        ],
        out_specs=pl.BlockSpec((BLK, H), lambda i: (i, 0)),
        out_shape=jax.ShapeDtypeStruct((EP, H), jnp.float32),
        interpret=_INTERP,
    )(eaT, elT, bm, ap, wbu, bbuT, wbf, bbond)


def _kd_body(np_ref, h_ref, gb_ref, wih_ref, whh_ref, bih_ref, bhh_ref, o_ref):
    x = jnp.maximum(np_ref[0] + np_ref[1] + gb_ref[...], 0.0)
    h = h_ref[...]
    gi = jnp.dot(x, wih_ref[...].T, preferred_element_type=jnp.float32) + bih_ref[...]
    gh = jnp.dot(h, whh_ref[...].T, preferred_element_type=jnp.float32) + bhh_ref[...]
    r = jax.nn.sigmoid(gi[:, :H] + gh[:, :H])
    z = jax.nn.sigmoid(gi[:, H:2 * H] + gh[:, H:2 * H])
    n = jnp.tanh(gi[:, 2 * H:] + r * gh[:, 2 * H:])
    o_ref[...] = (1.0 - z) * n + z * h


def _gru_step(npart, h_gru, gnn_bias, wih, whh, bih, bhh):
    return pl.pallas_call(
        _kd_body,
        grid=(NP // BLK,),
        in_specs=[
            pl.BlockSpec((2, BLK, H), lambda i: (0, i, 0)),
            pl.BlockSpec((BLK, H), lambda i: (i, 0)),
            pl.BlockSpec((1, H), lambda i: (0, 0)),
            pl.BlockSpec((3 * H, H), lambda i: (0, 0)),
            pl.BlockSpec((3 * H, H), lambda i: (0, 0)),
            pl.BlockSpec((1, 3 * H), lambda i: (0, 0)),
            pl.BlockSpec((1, 3 * H), lambda i: (0, 0)),
        ],
        out_specs=pl.BlockSpec((BLK, H), lambda i: (i, 0)),
        out_shape=jax.ShapeDtypeStruct((NP, H), jnp.float32),
        interpret=_INTERP,
    )(npart, h_gru, gnn_bias, wih, whh, bih, bhh)


def _lstm(x, h, c, wih, whh, bih, bhh):
    d = h.shape[-1]
    g = (jnp.dot(x, wih.T, preferred_element_type=jnp.float32) + bih
         + jnp.dot(h, whh.T, preferred_element_type=jnp.float32) + bhh)
    i = jax.nn.sigmoid(g[:, :d])
    f = jax.nn.sigmoid(g[:, d:2 * d])
    gg = jnp.tanh(g[:, 2 * d:3 * d])
    o = jax.nn.sigmoid(g[:, 3 * d:])
    c2 = f * c + i * gg
    return o * jnp.tanh(c2), c2


def _kz_body(x_ref, nf_ref, wih0_ref, whh0_ref, bih0_ref, bhh0_ref,
             wih1_ref, whh1_ref, bih1_ref, bhh1_ref, wsp_ref, bsp_ref, pa_ref,
             o_ref):
    na = jnp.concatenate([x_ref[...], nf_ref[...]], axis=1)
    rows = lax.broadcasted_iota(jnp.int32, (NP, 1), 0)
    valid = rows < N
    d = 2 * H
    q_star = jnp.zeros((1, 2 * d), jnp.float32)
    h0 = jnp.zeros((1, d), jnp.float32)
    c0 = jnp.zeros((1, d), jnp.float32)
    h1 = jnp.zeros((1, d), jnp.float32)
    c1 = jnp.zeros((1, d), jnp.float32)
    for _ in range(POOL_ITERS):
        h0, c0 = _lstm(q_star, h0, c0, wih0_ref[...], whh0_ref[...],
                       bih0_ref[...], bhh0_ref[...])
        h1, c1 = _lstm(h0, h1, c1, wih1_ref[...], whh1_ref[...],
                       bih1_ref[...], bhh1_ref[...])
        q = h1
        e = jnp.sum(na * q, axis=-1, keepdims=True)
        e = jnp.where(valid, e, -1e30)
        m = jnp.max(e, axis=0, keepdims=True)
        p = jnp.where(valid, jnp.exp(e - m), 0.0)
        alpha = p / jnp.sum(p, axis=0, keepdims=True)
        readout = jnp.sum(na * alpha, axis=0, keepdims=True)
        q_star = jnp.concatenate([q, readout], axis=-1)
    y = jnp.dot(q_star, wsp_ref[...], preferred_element_type=jnp.float32) + bsp_ref[...]
    o_ref[...] = jnp.where(y >= 0.0, y, pa_ref[...] * y)


def _set2set(x, nf, wih0, whh0, bih0, bhh0, wih1, whh1, bih1, bhh1, wsp, bsp, pa):
    full = lambda s: pl.BlockSpec(s, lambda: tuple(0 for _ in s))
    return pl.pallas_call(
        _kz_body,
        in_specs=[
            full((NP, H)), full((NP, H)),
            full((4 * 2 * H, 4 * H)), full((4 * 2 * H, 2 * H)),
            full((1, 4 * 2 * H)), full((1, 4 * 2 * H)),
            full((4 * 2 * H, 2 * H)), full((4 * 2 * H, 2 * H)),
            full((1, 4 * 2 * H)), full((1, 4 * 2 * H)),
            full((4 * H, D_HID)), full((1, D_HID)), full((1, 1)),
        ],
        out_specs=full((1, D_HID)),
        out_shape=jax.ShapeDtypeStruct((1, D_HID), jnp.float32),
        interpret=_INTERP,
    )(x, nf, wih0, whh0, bih0, bhh0, wih1, whh1, bih1, bhh1, wsp, bsp, pa)


_GRP = 13


def _sc_gather(table, idx3):
    _, C, _ = idx3.shape
    D = table.shape[1]
    groups = [(g, min(_GRP, C - g)) for g in range(0, C, _GRP)]
    mesh = plsc.VectorSubcoreMesh(core_axis_name="c", subcore_axis_name="s")

    @functools.partial(
        pl.kernel,
        out_type=jax.ShapeDtypeStruct((NW, C, CH, D), jnp.float32),
        mesh=mesh,
        compiler_params=pltpu.CompilerParams(use_tc_tiling_on_sc=False),
        scratch_types=[
            pltpu.VMEM((C, CH), jnp.int32),
            pltpu.VMEM((_GRP, CH, D), jnp.float32),
            pltpu.SemaphoreType.DMA,
        ],
    )
    def k(table_hbm, idx_hbm, out_hbm, idx_v, buf_v, sem):
        cid = lax.axis_index("c")
        sid = lax.axis_index("s")
        wid = sid * 2 + cid
        pltpu.sync_copy(idx_hbm.at[wid], idx_v)
        for g0, gsz in groups:
            cps = [pltpu.async_copy(table_hbm.at[idx_v.at[g0 + j]], buf_v.at[j], sem)
                   for j in range(gsz)]
            for cp in cps:
                cp.wait()
            pltpu.sync_copy(buf_v.at[pl.ds(0, gsz)], out_hbm.at[wid, pl.ds(g0, gsz)])

    return k(table, idx3)


def _gather_rows(table, idx_pad):
    B = idx_pad.shape[0]
    C = B // (NW * CH)
    out = _sc_gather(table, idx_pad.reshape(NW, C, CH))
    return out.reshape(B, table.shape[1])


def _sc_scatter_add(payload4, idx3, zeros, s_acc, s_out):
    _, C, _, D = payload4.shape
    rpt_i = s_acc // 16
    rpt_o = s_out // 16
    groups = [(g, min(_GRP, C - g)) for g in range(0, C, _GRP)]
    mesh = plsc.VectorSubcoreMesh(core_axis_name="c", subcore_axis_name="s")

    @functools.partial(
        pl.kernel,
        out_type=jax.ShapeDtypeStruct((2, s_out, D), jnp.float32),
        mesh=mesh,
        compiler_params=pltpu.CompilerParams(use_tc_tiling_on_sc=False),
        scratch_types=[
            pltpu.VMEM((C, CH), jnp.int32),
            pltpu.VMEM((_GRP, CH, D), jnp.float32),
            pltpu.VMEM_SHARED((s_acc, D), jnp.float32),
            pltpu.SemaphoreType.DMA,
        ],
    )
    def k(pay_hbm, idx_hbm, z_hbm, out_hbm, idx_v, buf_v, acc_sh, sem):
        cid = lax.axis_index("c")
        sid = lax.axis_index("s")
        wid = sid * 2 + cid
        pltpu.sync_copy(z_hbm.at[pl.ds(sid * rpt_i, rpt_i)],
                        acc_sh.at[pl.ds(sid * rpt_i, rpt_i)])
        pltpu.sync_copy(idx_hbm.at[wid], idx_v)
        plsc.subcore_barrier()
        for g0, gsz in groups:
            pltpu.sync_copy(pay_hbm.at[wid, pl.ds(g0, gsz)], buf_v.at[pl.ds(0, gsz)])
            cps = [pltpu.async_copy(buf_v.at[j], acc_sh.at[idx_v.at[g0 + j]],
                                    sem, add=True)
                   for j in range(gsz)]
            for cp in cps:
                cp.wait()
        plsc.subcore_barrier()
        pltpu.sync_copy(acc_sh.at[pl.ds(sid * rpt_o, rpt_o)],
                        out_hbm.at[cid, pl.ds(sid * rpt_o, rpt_o)])

    return k(payload4, idx3, zeros)


def _scatter_add(payload, idx_pad, nseg):
    B, D = payload.shape
    C = B // (NW * CH)
    pay4 = payload.reshape(NW, C, CH, D)
    if nseg * D <= 1024 * 1024:
        zeros = jnp.zeros((nseg, D), jnp.float32)
        return _sc_scatter_add(pay4, idx_pad.reshape(NW, C, CH), zeros,
                               nseg, nseg)
    hp = nseg // 2
    s_acc = hp + 128
    zeros = jnp.zeros((s_acc, D), jnp.float32)
    idx_a = jnp.where(idx_pad < hp, idx_pad, hp).reshape(NW, C, CH)
    idx_b = jnp.where(idx_pad >= hp, idx_pad - hp, hp).reshape(NW, C, CH)
    out_a = _sc_scatter_add(pay4, idx_a, zeros, s_acc, hp)
    out_b = _sc_scatter_add(pay4, idx_b, zeros, s_acc, hp)
    return jnp.stack([out_a, out_b], axis=1).reshape(2, nseg, D)


def kernel(node_attr, edge_index, edge_attr, edge_length, ee_index, ee_angle,
           W_proj, b_proj, W_bond, b_bond, W_edgefn, b_edgefn, W_bu, b_bu,
           W_au, b_au, gnn_bias, gru_Wih, gru_Whh, gru_bih, gru_bhh,
           s2s_Wih0, s2s_Whh0, s2s_bih0, s2s_bhh0,
           s2s_Wih1, s2s_Whh1, s2s_bih1, s2s_bhh1, W_sp, b_sp, prelu_a):
    f32 = jnp.float32
    src = edge_index[0].astype(jnp.int32)
    dst = edge_index[1].astype(jnp.int32)
    ee_src = ee_index[0].astype(jnp.int32)
    ee_dst = ee_index[1].astype(jnp.int32)

    na_pad = jnp.pad(node_attr, ((0, NP - N), (0, 112 - D_NODE)))
    wp_pad = jnp.pad(W_proj, ((0, 112 - D_NODE), (0, 0)))
    w2f = W_edgefn.reshape(16, H, H).transpose(1, 0, 2).reshape(H, 16 * H)
    bline = b_edgefn.reshape(H, H)
    wbf = W_bond.reshape(16, H, H).transpose(1, 0, 2).reshape(H, 16 * H)
    bbond = b_bond.reshape(H, H)

    ee_src_p = jnp.pad(ee_src, (0, E2P - E2))
    ee_dst_p = jnp.pad(ee_dst, (0, E2P - E2))
    dst_p = jnp.pad(dst, (0, EP - E))
    src_p = jnp.pad(src, (0, EP - E))
    angT = jnp.pad(ee_angle, (0, E2P - E2))[None, :]
    eaT = jnp.pad(edge_attr, ((0, EP - E), (0, 0))).T
    elT = jnp.pad(edge_length, (0, EP - E))[None, :]

    src_bits = lax.bitcast_convert_type(src, f32)
    table16 = jnp.zeros((E, 16), f32)
    table16 = table16.at[:, 0].set(edge_length)
    table16 = table16.at[:, 1].set(src_bits)

    g2 = _gather_rows(table16, ee_src_p)
    src2_p = lax.bitcast_convert_type(g2[:, 1], jnp.int32)
    el2T = g2[:, 0][None, :]
    idxg = jnp.concatenate([src_p, src2_p])

    nf = _node_proj(na_pad, wp_pad, b_proj[None])

    x = nf
    h_gru = nf
    for _ in range(STEPS):
        g = _gather_rows(x, idxg)
        bm = g[:EP]
        bm2 = g[EP:]
        out_line = _line_msg(angT, el2T, bm2, W_au, b_au[:, None], w2f, bline)
        ap = _scatter_add(out_line, ee_dst_p, EP)
        m = _bond_msg(eaT, elT, bm, ap, W_bu, b_bu[:, None], wbf, bbond)
        npart = _scatter_add(m, dst_p, NP)
        x = _gru_step(npart, h_gru, gnn_bias[None], gru_Wih, gru_Whh,
                      gru_bih[None], gru_bhh[None])
        h_gru = x

    return _set2set(x, nf, s2s_Wih0, s2s_Whh0, s2s_bih0[None], s2s_bhh0[None],
                    s2s_Wih1, s2s_Whh1, s2s_bih1[None], s2s_bhh1[None],
                    W_sp, b_sp[None], prelu_a.reshape(1, 1))


# speedup vs baseline: 22.2132x; 1.0712x over previous
import functools

import jax
import jax.numpy as jnp
from jax import lax
from jax.experimental import pallas as pl
from jax.experimental.pallas import tpu as pltpu
from jax.experimental.pallas import tpu_sc as plsc

N = 20000
E = 50000
E2 = 100000
D_NODE = 110
D_EATTR = 8
K = 8
H = 32
D_HID = 4096
STEPS = 2
POOL_ITERS = 3

CUTOFF = 5.0
GAMMA = 10.0
CENTERS = [CUTOFF * i / (K - 1) for i in range(K)]

NW = 32
CH = 128
C_E = 13
C_E2 = 25
C_G = C_E + C_E2
EP = NW * C_E * CH
E2P = NW * C_E2 * CH
BG = NW * C_G * CH
NP = 20480
BLK = 2048

_INTERP = False


def _rbf_col(d, k):
    return jnp.exp(-GAMMA * (d - CENTERS[k]) ** 2)


def _k0_body(na_ref, wp_ref, bp_ref, o_ref):
    i = pl.program_id(0)
    x = jnp.maximum(jnp.dot(na_ref[...], wp_ref[...],
                            preferred_element_type=jnp.float32) + bp_ref[...], 0.0)
    rows = i * BLK + lax.broadcasted_iota(jnp.int32, (BLK, 1), 0)
    o_ref[...] = jnp.where(rows < N, x, 0.0)


def _node_proj(na_pad, wp_pad, bp):
    return pl.pallas_call(
        _k0_body,
        grid=(NP // BLK,),
        in_specs=[
            pl.BlockSpec((BLK, 112), lambda i: (i, 0)),
            pl.BlockSpec((112, H), lambda i: (0, 0)),
            pl.BlockSpec((1, H), lambda i: (0, 0)),
        ],
        out_specs=pl.BlockSpec((BLK, H), lambda i: (i, 0)),
        out_shape=jax.ShapeDtypeStruct((NP, H), jnp.float32),
        interpret=_INTERP,
    )(na_pad, wp_pad, bp)


def _dotT(a, b):
    return lax.dot_general(a, b, (((0,), (1,)), ((), ())),
                           preferred_element_type=jnp.float32)


def _dot00(a, b):
    return lax.dot_general(a, b, (((0,), (0,)), ((), ())),
                           preferred_element_type=jnp.float32)


def _kb_body(angT_ref, elT_ref, bm2_ref, wau_ref, bauT_ref, w2f_ref, bl_ref, o_ref):
    i = pl.program_id(0)
    hlT = jnp.maximum(_dotT(wau_ref[...], bm2_ref[...]) + bauT_ref[...], 0.0)
    yT = _dot00(w2f_ref[...], hlT)
    accT = _dot00(bl_ref[...], hlT)
    angT = angT_ref[...]
    elT = elT_ref[...]
    for k in range(K):
        accT += yT[k * H:(k + 1) * H] * _rbf_col(angT, k)
    for k in range(K):
        accT += yT[(K + k) * H:(K + k + 1) * H] * _rbf_col(elT, k)
    rows = i * BLK + lax.broadcasted_iota(jnp.int32, (BLK, 1), 0)
    o_ref[...] = jnp.where(rows < E2, accT.T, 0.0)


def _line_msg(angT, el2T, bm2, wau, bauT, w2f, bline):
    return pl.pallas_call(
        _kb_body,
        grid=(E2P // BLK,),
        in_specs=[
            pl.BlockSpec((1, BLK), lambda i: (0, i)),
            pl.BlockSpec((1, BLK), lambda i: (0, i)),
            pl.BlockSpec((BLK, H), lambda i: (i, 0)),
            pl.BlockSpec((H, H), lambda i: (0, 0)),
            pl.BlockSpec((H, 1), lambda i: (0, 0)),
            pl.BlockSpec((H, 16 * H), lambda i: (0, 0)),
            pl.BlockSpec((H, H), lambda i: (0, 0)),
        ],
        out_specs=pl.BlockSpec((BLK, H), lambda i: (i, 0)),
        out_shape=jax.ShapeDtypeStruct((E2P, H), jnp.float32),
        interpret=_INTERP,
    )(angT, el2T, bm2, wau, bauT, w2f, bline)


def _kc_body(eaT_ref, elT_ref, bm_ref, ap_ref, wbu_ref, bbuT_ref, wbf_ref, bb_ref, o_ref):
    i = pl.program_id(0)
    hbT = jnp.maximum(_dotT(wbu_ref[...], bm_ref[...]) + bbuT_ref[...], 0.0)
    hbT = hbT + (ap_ref[0] + ap_ref[1]).T
    yT = _dot00(wbf_ref[...], hbT)
    accT = _dot00(bb_ref[...], hbT)
    eaT = eaT_ref[...]
    elT = elT_ref[...]
    for k in range(D_EATTR):
        accT += yT[k * H:(k + 1) * H] * eaT[k:k + 1]
    for k in range(K):
        accT += yT[(D_EATTR + k) * H:(D_EATTR + k + 1) * H] * _rbf_col(elT, k)
    rows = i * BLK + lax.broadcasted_iota(jnp.int32, (BLK, 1), 0)
    o_ref[...] = jnp.where(rows < E, accT.T, 0.0)


def _bond_msg(eaT, elT, bm, ap, wbu, bbuT, wbf, bbond):
    return pl.pallas_call(
        _kc_body,
        grid=(EP // BLK,),
        in_specs=[
            pl.BlockSpec((D_EATTR, BLK), lambda i: (0, i)),
            pl.BlockSpec((1, BLK), lambda i: (0, i)),
            pl.BlockSpec((BLK, H), lambda i: (i, 0)),
            pl.BlockSpec((2, BLK, H), lambda i: (0, i, 0)),
            pl.BlockSpec((H, H), lambda i: (0, 0)),
            pl.BlockSpec((H, 1), lambda i: (0, 0)),
            pl.BlockSpec((H, 16 * H), lambda i: (0, 0)),
            pl.BlockSpec((H, H), lambda i: (0, 0)),
        ],
        out_specs=pl.BlockSpec((BLK, H), lambda i: (i, 0)),
        out_shape=jax.ShapeDtypeStruct((EP, H), jnp.float32),
        interpret=_INTERP,
    )(eaT, elT, bm, ap, wbu, bbuT, wbf, bbond)


def _kd_body(np_ref, h_ref, gb_ref, wih_ref, whh_ref, bih_ref, bhh_ref, o_ref):
    x = jnp.maximum(np_ref[0] + np_ref[1] + gb_ref[...], 0.0)
    h = h_ref[...]
    gi = jnp.dot(x, wih_ref[...].T, preferred_element_type=jnp.float32) + bih_ref[...]
    gh = jnp.dot(h, whh_ref[...].T, preferred_element_type=jnp.float32) + bhh_ref[...]
    r = jax.nn.sigmoid(gi[:, :H] + gh[:, :H])
    z = jax.nn.sigmoid(gi[:, H:2 * H] + gh[:, H:2 * H])
    n = jnp.tanh(gi[:, 2 * H:] + r * gh[:, 2 * H:])
    o_ref[...] = (1.0 - z) * n + z * h


def _gru_step(npart, h_gru, gnn_bias, wih, whh, bih, bhh):
    return pl.pallas_call(
        _kd_body,
        grid=(NP // BLK,),
        in_specs=[
            pl.BlockSpec((2, BLK, H), lambda i: (0, i, 0)),
            pl.BlockSpec((BLK, H), lambda i: (i, 0)),
            pl.BlockSpec((1, H), lambda i: (0, 0)),
            pl.BlockSpec((3 * H, H), lambda i: (0, 0)),
            pl.BlockSpec((3 * H, H), lambda i: (0, 0)),
            pl.BlockSpec((1, 3 * H), lambda i: (0, 0)),
            pl.BlockSpec((1, 3 * H), lambda i: (0, 0)),
        ],
        out_specs=pl.BlockSpec((BLK, H), lambda i: (i, 0)),
        out_shape=jax.ShapeDtypeStruct((NP, H), jnp.float32),
        interpret=_INTERP,
    )(npart, h_gru, gnn_bias, wih, whh, bih, bhh)


def _lstm(x, h, c, wih, whh, bih, bhh):
    d = h.shape[-1]
    g = (jnp.dot(x, wih.T, preferred_element_type=jnp.float32) + bih
         + jnp.dot(h, whh.T, preferred_element_type=jnp.float32) + bhh)
    i = jax.nn.sigmoid(g[:, :d])
    f = jax.nn.sigmoid(g[:, d:2 * d])
    gg = jnp.tanh(g[:, 2 * d:3 * d])
    o = jax.nn.sigmoid(g[:, 3 * d:])
    c2 = f * c + i * gg
    return o * jnp.tanh(c2), c2


def _kz_body(x_ref, nf_ref, wih0_ref, whh0_ref, bih0_ref, bhh0_ref,
             wih1_ref, whh1_ref, bih1_ref, bhh1_ref, wsp_ref, bsp_ref, pa_ref,
             o_ref):
    na = jnp.concatenate([x_ref[...], nf_ref[...]], axis=1)
    rows = lax.broadcasted_iota(jnp.int32, (NP, 1), 0)
    valid = rows < N
    d = 2 * H
    q_star = jnp.zeros((1, 2 * d), jnp.float32)
    h0 = jnp.zeros((1, d), jnp.float32)
    c0 = jnp.zeros((1, d), jnp.float32)
    h1 = jnp.zeros((1, d), jnp.float32)
    c1 = jnp.zeros((1, d), jnp.float32)
    for _ in range(POOL_ITERS):
        h0, c0 = _lstm(q_star, h0, c0, wih0_ref[...], whh0_ref[...],
                       bih0_ref[...], bhh0_ref[...])
        h1, c1 = _lstm(h0, h1, c1, wih1_ref[...], whh1_ref[...],
                       bih1_ref[...], bhh1_ref[...])
        q = h1
        e = jnp.sum(na * q, axis=-1, keepdims=True)
        e = jnp.where(valid, e, -1e30)
        m = jnp.max(e, axis=0, keepdims=True)
        p = jnp.where(valid, jnp.exp(e - m), 0.0)
        alpha = p / jnp.sum(p, axis=0, keepdims=True)
        readout = jnp.sum(na * alpha, axis=0, keepdims=True)
        q_star = jnp.concatenate([q, readout], axis=-1)
    y = jnp.dot(q_star, wsp_ref[...], preferred_element_type=jnp.float32) + bsp_ref[...]
    o_ref[...] = jnp.where(y >= 0.0, y, pa_ref[...] * y)


def _set2set(x, nf, wih0, whh0, bih0, bhh0, wih1, whh1, bih1, bhh1, wsp, bsp, pa):
    full = lambda s: pl.BlockSpec(s, lambda: tuple(0 for _ in s))
    return pl.pallas_call(
        _kz_body,
        in_specs=[
            full((NP, H)), full((NP, H)),
            full((4 * 2 * H, 4 * H)), full((4 * 2 * H, 2 * H)),
            full((1, 4 * 2 * H)), full((1, 4 * 2 * H)),
            full((4 * 2 * H, 2 * H)), full((4 * 2 * H, 2 * H)),
            full((1, 4 * 2 * H)), full((1, 4 * 2 * H)),
            full((4 * H, D_HID)), full((1, D_HID)), full((1, 1)),
        ],
        out_specs=full((1, D_HID)),
        out_shape=jax.ShapeDtypeStruct((1, D_HID), jnp.float32),
        interpret=_INTERP,
    )(x, nf, wih0, whh0, bih0, bhh0, wih1, whh1, bih1, bhh1, wsp, bsp, pa)


_GRP = 13


def _sc_gather(table, idx3):
    _, C, _ = idx3.shape
    D = table.shape[1]
    groups = [(g, min(_GRP, C - g)) for g in range(0, C, _GRP)]
    mesh = plsc.VectorSubcoreMesh(core_axis_name="c", subcore_axis_name="s")

    @functools.partial(
        pl.kernel,
        out_type=jax.ShapeDtypeStruct((NW, C, CH, D), jnp.float32),
        mesh=mesh,
        compiler_params=pltpu.CompilerParams(use_tc_tiling_on_sc=False),
        scratch_types=[
            pltpu.VMEM((C, CH), jnp.int32),
            pltpu.VMEM((2, _GRP, CH, D), jnp.float32),
            pltpu.SemaphoreType.DMA,
            pltpu.SemaphoreType.DMA,
            pltpu.SemaphoreType.DMA,
        ],
    )
    def k(table_hbm, idx_hbm, out_hbm, idx_v, buf_v, sem_g, sem_w0, sem_w1):
        cid = lax.axis_index("c")
        sid = lax.axis_index("s")
        wid = sid * 2 + cid
        pltpu.sync_copy(idx_hbm.at[wid], idx_v)
        sem_w = [sem_w0, sem_w1]
        wr = [None, None]
        for gi, (g0, gsz) in enumerate(groups):
            b = gi % 2
            if wr[b] is not None:
                wr[b].wait()
            cps = [pltpu.async_copy(table_hbm.at[idx_v.at[g0 + j]],
                                    buf_v.at[b, j], sem_g)
                   for j in range(gsz)]
            for cp in cps:
                cp.wait()
            wr[b] = pltpu.async_copy(buf_v.at[b, pl.ds(0, gsz)],
                                     out_hbm.at[wid, pl.ds(g0, gsz)], sem_w[b])
        for w in wr:
            if w is not None:
                w.wait()

    return k(table, idx3)


def _gather_rows(table, idx_pad):
    B = idx_pad.shape[0]
    C = B // (NW * CH)
    out = _sc_gather(table, idx_pad.reshape(NW, C, CH))
    return out.reshape(B, table.shape[1])


def _sc_scatter_add(payload4, idx3, zeros, S, pd):
    _, C, _, D = payload4.shape
    rpt = S // 16
    groups = [(g, min(_GRP, C - g)) for g in range(0, C, _GRP)]
    phases = [(c0, pd) for c0 in range(0, D, pd)]
    mesh = plsc.VectorSubcoreMesh(core_axis_name="c", subcore_axis_name="s")

    @functools.partial(
        pl.kernel,
        out_type=jax.ShapeDtypeStruct((2, S, D), jnp.float32),
        mesh=mesh,
        compiler_params=pltpu.CompilerParams(use_tc_tiling_on_sc=False),
        scratch_types=[
            pltpu.VMEM((C, CH), jnp.int32),
            pltpu.VMEM((2, _GRP, CH, pd), jnp.float32),
            pltpu.VMEM_SHARED((S, pd), jnp.float32),
            pltpu.SemaphoreType.DMA,
            pltpu.SemaphoreType.DMA,
            pltpu.SemaphoreType.DMA,
        ],
    )
    def k(pay_hbm, idx_hbm, z_hbm, out_hbm, idx_v, buf_v, acc_sh,
          sem_a, sem_l0, sem_l1):
        cid = lax.axis_index("c")
        sid = lax.axis_index("s")
        wid = sid * 2 + cid
        pltpu.sync_copy(idx_hbm.at[wid], idx_v)
        sem_l = [sem_l0, sem_l1]
        for c0, _ in phases:
            pltpu.sync_copy(z_hbm.at[pl.ds(sid * rpt, rpt)],
                            acc_sh.at[pl.ds(sid * rpt, rpt)])
            plsc.subcore_barrier()
            g0, gsz = groups[0]
            ld = [None, None]
            ld[0] = pltpu.async_copy(
                pay_hbm.at[wid, pl.ds(g0, gsz), :, pl.ds(c0, pd)],
                buf_v.at[0, pl.ds(0, gsz)], sem_l[0])
            for gi, (g0, gsz) in enumerate(groups):
                b = gi % 2
                ld[b].wait()
                if gi + 1 < len(groups):
                    n0, nsz = groups[gi + 1]
                    nb = (gi + 1) % 2
                    ld[nb] = pltpu.async_copy(
                        pay_hbm.at[wid, pl.ds(n0, nsz), :, pl.ds(c0, pd)],
                        buf_v.at[nb, pl.ds(0, nsz)], sem_l[nb])
                cps = [pltpu.async_copy(buf_v.at[b, j],
                                        acc_sh.at[idx_v.at[g0 + j]],
                                        sem_a, add=True)
                       for j in range(gsz)]
                for cp in cps:
                    cp.wait()
            plsc.subcore_barrier()
            pltpu.sync_copy(acc_sh.at[pl.ds(sid * rpt, rpt)],
                            out_hbm.at[cid, pl.ds(sid * rpt, rpt), pl.ds(c0, pd)])
            plsc.subcore_barrier()

    return k(payload4, idx3, zeros)


def _scatter_add(payload, idx_pad, nseg):
    B, D = payload.shape
    C = B // (NW * CH)
    pay4 = payload.reshape(NW, C, CH, D)
    pd = D if nseg * D <= 600 * 1024 else D // 2
    zeros = jnp.zeros((nseg, pd), jnp.float32)
    return _sc_scatter_add(pay4, idx_pad.reshape(NW, C, CH), zeros, nseg, pd)


def kernel(node_attr, edge_index, edge_attr, edge_length, ee_index, ee_angle,
           W_proj, b_proj, W_bond, b_bond, W_edgefn, b_edgefn, W_bu, b_bu,
           W_au, b_au, gnn_bias, gru_Wih, gru_Whh, gru_bih, gru_bhh,
           s2s_Wih0, s2s_Whh0, s2s_bih0, s2s_bhh0,
           s2s_Wih1, s2s_Whh1, s2s_bih1, s2s_bhh1, W_sp, b_sp, prelu_a):
    f32 = jnp.float32
    src = edge_index[0].astype(jnp.int32)
    dst = edge_index[1].astype(jnp.int32)
    ee_src = ee_index[0].astype(jnp.int32)
    ee_dst = ee_index[1].astype(jnp.int32)

    na_pad = jnp.pad(node_attr, ((0, NP - N), (0, 112 - D_NODE)))
    wp_pad = jnp.pad(W_proj, ((0, 112 - D_NODE), (0, 0)))
    w2f = W_edgefn.reshape(16, H, H).transpose(1, 0, 2).reshape(H, 16 * H)
    bline = b_edgefn.reshape(H, H)
    wbf = W_bond.reshape(16, H, H).transpose(1, 0, 2).reshape(H, 16 * H)
    bbond = b_bond.reshape(H, H)

    ee_src_p = jnp.pad(ee_src, (0, E2P - E2))
    ee_dst_p = jnp.pad(ee_dst, (0, E2P - E2))
    dst_p = jnp.pad(dst, (0, EP - E))
    src_p = jnp.pad(src, (0, EP - E))
    angT = jnp.pad(ee_angle, (0, E2P - E2))[None, :]
    eaT = jnp.pad(edge_attr, ((0, EP - E), (0, 0))).T
    elT = jnp.pad(edge_length, (0, EP - E))[None, :]

    src_bits = lax.bitcast_convert_type(src, f32)
    table16 = jnp.zeros((E, 16), f32)
    table16 = table16.at[:, 0].set(edge_length)
    table16 = table16.at[:, 1].set(src_bits)

    g2 = _gather_rows(table16, ee_src_p)
    src2_p = lax.bitcast_convert_type(g2[:, 1], jnp.int32)
    el2T = g2[:, 0][None, :]
    idxg = jnp.concatenate([src_p, src2_p])

    nf = _node_proj(na_pad, wp_pad, b_proj[None])

    x = nf
    h_gru = nf
    for _ in range(STEPS):
        g = _gather_rows(x, idxg)
        bm = g[:EP]
        bm2 = g[EP:]
        out_line = _line_msg(angT, el2T, bm2, W_au, b_au[:, None], w2f, bline)
        ap = _scatter_add(out_line, ee_dst_p, EP)
        m = _bond_msg(eaT, elT, bm, ap, W_bu, b_bu[:, None], wbf, bbond)
        npart = _scatter_add(m, dst_p, NP)
        x = _gru_step(npart, h_gru, gnn_bias[None], gru_Wih, gru_Whh,
                      gru_bih[None], gru_bhh[None])
        h_gru = x

    return _set2set(x, nf, s2s_Wih0, s2s_Whh0, s2s_bih0[None], s2s_bhh0[None],
                    s2s_Wih1, s2s_Whh1, s2s_bih1[None], s2s_bhh1[None],
                    W_sp, b_sp[None], prelu_a.reshape(1, 1))


# speedup vs baseline: 22.2297x; 1.0007x over previous
import functools

import jax
import jax.numpy as jnp
from jax import lax
from jax.experimental import pallas as pl
from jax.experimental.pallas import tpu as pltpu
from jax.experimental.pallas import tpu_sc as plsc

N = 20000
E = 50000
E2 = 100000
D_NODE = 110
D_EATTR = 8
K = 8
H = 32
D_HID = 4096
STEPS = 2
POOL_ITERS = 3

CUTOFF = 5.0
GAMMA = 10.0
CENTERS = [CUTOFF * i / (K - 1) for i in range(K)]

NW = 32
CH = 128
C_E = 13
C_E2 = 25
C_G = C_E + C_E2
EP = NW * C_E * CH
E2P = NW * C_E2 * CH
BG = NW * C_G * CH
NP = 20480
BLK = 2048

_INTERP = False


def _rbf_col(d, k):
    return jnp.exp(-GAMMA * (d - CENTERS[k]) ** 2)


def _k0_body(na_ref, wp_ref, bp_ref, o_ref):
    i = pl.program_id(0)
    x = jnp.maximum(jnp.dot(na_ref[...], wp_ref[...],
                            preferred_element_type=jnp.float32) + bp_ref[...], 0.0)
    rows = i * BLK + lax.broadcasted_iota(jnp.int32, (BLK, 1), 0)
    o_ref[...] = jnp.where(rows < N, x, 0.0)


def _node_proj(na_pad, wp_pad, bp):
    return pl.pallas_call(
        _k0_body,
        grid=(NP // BLK,),
        in_specs=[
            pl.BlockSpec((BLK, 112), lambda i: (i, 0)),
            pl.BlockSpec((112, H), lambda i: (0, 0)),
            pl.BlockSpec((1, H), lambda i: (0, 0)),
        ],
        out_specs=pl.BlockSpec((BLK, H), lambda i: (i, 0)),
        out_shape=jax.ShapeDtypeStruct((NP, H), jnp.float32),
        interpret=_INTERP,
    )(na_pad, wp_pad, bp)


def _dotT(a, b):
    return lax.dot_general(a, b, (((0,), (1,)), ((), ())),
                           preferred_element_type=jnp.float32)


def _dot00(a, b):
    return lax.dot_general(a, b, (((0,), (0,)), ((), ())),
                           preferred_element_type=jnp.float32)


def _kb_body(angT_ref, elT_ref, bm2_ref, wau_ref, bauT_ref, w2f_ref, bl_ref, o_ref):
    i = pl.program_id(0)
    hlT = jnp.maximum(_dotT(wau_ref[...], bm2_ref[...]) + bauT_ref[...], 0.0)
    yT = _dot00(w2f_ref[...], hlT)
    accT = _dot00(bl_ref[...], hlT)
    angT = angT_ref[...]
    elT = elT_ref[...]
    for k in range(K):
        accT += yT[k * H:(k + 1) * H] * _rbf_col(angT, k)
    for k in range(K):
        accT += yT[(K + k) * H:(K + k + 1) * H] * _rbf_col(elT, k)
    rows = i * BLK + lax.broadcasted_iota(jnp.int32, (BLK, 1), 0)
    o_ref[...] = jnp.where(rows < E2, accT.T, 0.0)


def _line_msg(angT, el2T, bm2, wau, bauT, w2f, bline):
    return pl.pallas_call(
        _kb_body,
        grid=(E2P // BLK,),
        in_specs=[
            pl.BlockSpec((1, BLK), lambda i: (0, i)),
            pl.BlockSpec((1, BLK), lambda i: (0, i)),
            pl.BlockSpec((BLK, H), lambda i: (i, 0)),
            pl.BlockSpec((H, H), lambda i: (0, 0)),
            pl.BlockSpec((H, 1), lambda i: (0, 0)),
            pl.BlockSpec((H, 16 * H), lambda i: (0, 0)),
            pl.BlockSpec((H, H), lambda i: (0, 0)),
        ],
        out_specs=pl.BlockSpec((BLK, H), lambda i: (i, 0)),
        out_shape=jax.ShapeDtypeStruct((E2P, H), jnp.float32),
        interpret=_INTERP,
    )(angT, el2T, bm2, wau, bauT, w2f, bline)


def _kc_body(eaT_ref, elT_ref, bm_ref, ap_ref, wbu_ref, bbuT_ref, wbf_ref, bb_ref, o_ref):
    i = pl.program_id(0)
    hbT = jnp.maximum(_dotT(wbu_ref[...], bm_ref[...]) + bbuT_ref[...], 0.0)
    hbT = hbT + (ap_ref[0] + ap_ref[1]).T
    yT = _dot00(wbf_ref[...], hbT)
    accT = _dot00(bb_ref[...], hbT)
    eaT = eaT_ref[...]
    elT = elT_ref[...]
    for k in range(D_EATTR):
        accT += yT[k * H:(k + 1) * H] * eaT[k:k + 1]
    for k in range(K):
        accT += yT[(D_EATTR + k) * H:(D_EATTR + k + 1) * H] * _rbf_col(elT, k)
    rows = i * BLK + lax.broadcasted_iota(jnp.int32, (BLK, 1), 0)
    o_ref[...] = jnp.where(rows < E, accT.T, 0.0)


def _bond_msg(eaT, elT, bm, ap, wbu, bbuT, wbf, bbond):
    return pl.pallas_call(
        _kc_body,
        grid=(EP // BLK,),
        in_specs=[
            pl.BlockSpec((D_EATTR, BLK), lambda i: (0, i)),
            pl.BlockSpec((1, BLK), lambda i: (0, i)),
            pl.BlockSpec((BLK, H), lambda i: (i, 0)),
            pl.BlockSpec((2, BLK, H), lambda i: (0, i, 0)),
            pl.BlockSpec((H, H), lambda i: (0, 0)),
            pl.BlockSpec((H, 1), lambda i: (0, 0)),
            pl.BlockSpec((H, 16 * H), lambda i: (0, 0)),
            pl.BlockSpec((H, H), lambda i: (0, 0)),
        ],
        out_specs=pl.BlockSpec((BLK, H), lambda i: (i, 0)),
        out_shape=jax.ShapeDtypeStruct((EP, H), jnp.float32),
        interpret=_INTERP,
    )(eaT, elT, bm, ap, wbu, bbuT, wbf, bbond)


def _kd_body(np_ref, h_ref, gb_ref, wih_ref, whh_ref, bih_ref, bhh_ref, o_ref):
    x = jnp.maximum(np_ref[0] + np_ref[1] + gb_ref[...], 0.0)
    h = h_ref[...]
    gi = jnp.dot(x, wih_ref[...].T, preferred_element_type=jnp.float32) + bih_ref[...]
    gh = jnp.dot(h, whh_ref[...].T, preferred_element_type=jnp.float32) + bhh_ref[...]
    r = jax.nn.sigmoid(gi[:, :H] + gh[:, :H])
    z = jax.nn.sigmoid(gi[:, H:2 * H] + gh[:, H:2 * H])
    n = jnp.tanh(gi[:, 2 * H:] + r * gh[:, 2 * H:])
    o_ref[...] = (1.0 - z) * n + z * h


def _gru_step(npart, h_gru, gnn_bias, wih, whh, bih, bhh):
    return pl.pallas_call(
        _kd_body,
        grid=(NP // BLK,),
        in_specs=[
            pl.BlockSpec((2, BLK, H), lambda i: (0, i, 0)),
            pl.BlockSpec((BLK, H), lambda i: (i, 0)),
            pl.BlockSpec((1, H), lambda i: (0, 0)),
            pl.BlockSpec((3 * H, H), lambda i: (0, 0)),
            pl.BlockSpec((3 * H, H), lambda i: (0, 0)),
            pl.BlockSpec((1, 3 * H), lambda i: (0, 0)),
            pl.BlockSpec((1, 3 * H), lambda i: (0, 0)),
        ],
        out_specs=pl.BlockSpec((BLK, H), lambda i: (i, 0)),
        out_shape=jax.ShapeDtypeStruct((NP, H), jnp.float32),
        interpret=_INTERP,
    )(npart, h_gru, gnn_bias, wih, whh, bih, bhh)


def _lstm(x, h, c, wih, whh, bih, bhh):
    d = h.shape[-1]
    g = (jnp.dot(x, wih.T, preferred_element_type=jnp.float32) + bih
         + jnp.dot(h, whh.T, preferred_element_type=jnp.float32) + bhh)
    i = jax.nn.sigmoid(g[:, :d])
    f = jax.nn.sigmoid(g[:, d:2 * d])
    gg = jnp.tanh(g[:, 2 * d:3 * d])
    o = jax.nn.sigmoid(g[:, 3 * d:])
    c2 = f * c + i * gg
    return o * jnp.tanh(c2), c2


def _kz_body(x_ref, nf_ref, wih0_ref, whh0_ref, bih0_ref, bhh0_ref,
             wih1_ref, whh1_ref, bih1_ref, bhh1_ref, wsp_ref, bsp_ref, pa_ref,
             o_ref):
    na = jnp.concatenate([x_ref[...], nf_ref[...]], axis=1)
    rows = lax.broadcasted_iota(jnp.int32, (NP, 1), 0)
    valid = rows < N
    d = 2 * H
    q_star = jnp.zeros((1, 2 * d), jnp.float32)
    h0 = jnp.zeros((1, d), jnp.float32)
    c0 = jnp.zeros((1, d), jnp.float32)
    h1 = jnp.zeros((1, d), jnp.float32)
    c1 = jnp.zeros((1, d), jnp.float32)
    for _ in range(POOL_ITERS):
        h0, c0 = _lstm(q_star, h0, c0, wih0_ref[...], whh0_ref[...],
                       bih0_ref[...], bhh0_ref[...])
        h1, c1 = _lstm(h0, h1, c1, wih1_ref[...], whh1_ref[...],
                       bih1_ref[...], bhh1_ref[...])
        q = h1
        e = jnp.sum(na * q, axis=-1, keepdims=True)
        e = jnp.where(valid, e, -1e30)
        m = jnp.max(e, axis=0, keepdims=True)
        p = jnp.where(valid, jnp.exp(e - m), 0.0)
        alpha = p / jnp.sum(p, axis=0, keepdims=True)
        readout = jnp.sum(na * alpha, axis=0, keepdims=True)
        q_star = jnp.concatenate([q, readout], axis=-1)
    y = jnp.dot(q_star, wsp_ref[...], preferred_element_type=jnp.float32) + bsp_ref[...]
    o_ref[...] = jnp.where(y >= 0.0, y, pa_ref[...] * y)


def _set2set(x, nf, wih0, whh0, bih0, bhh0, wih1, whh1, bih1, bhh1, wsp, bsp, pa):
    full = lambda s: pl.BlockSpec(s, lambda: tuple(0 for _ in s))
    return pl.pallas_call(
        _kz_body,
        in_specs=[
            full((NP, H)), full((NP, H)),
            full((4 * 2 * H, 4 * H)), full((4 * 2 * H, 2 * H)),
            full((1, 4 * 2 * H)), full((1, 4 * 2 * H)),
            full((4 * 2 * H, 2 * H)), full((4 * 2 * H, 2 * H)),
            full((1, 4 * 2 * H)), full((1, 4 * 2 * H)),
            full((4 * H, D_HID)), full((1, D_HID)), full((1, 1)),
        ],
        out_specs=full((1, D_HID)),
        out_shape=jax.ShapeDtypeStruct((1, D_HID), jnp.float32),
        interpret=_INTERP,
    )(x, nf, wih0, whh0, bih0, bhh0, wih1, whh1, bih1, bhh1, wsp, bsp, pa)


_GRP = 13


def _sc_gather(table, idx2):
    _, R = idx2.shape
    D = table.shape[1]
    GR = _GRP * CH
    groups = [(g, min(GR, R - g)) for g in range(0, R, GR)]
    mesh = plsc.VectorSubcoreMesh(core_axis_name="c", subcore_axis_name="s")

    @functools.partial(
        pl.kernel,
        out_type=jax.ShapeDtypeStruct((NW, R, D), jnp.float32),
        mesh=mesh,
        compiler_params=pltpu.CompilerParams(use_tc_tiling_on_sc=False),
        scratch_types=[
            pltpu.VMEM((R,), jnp.int32),
            pltpu.VMEM((2, GR, D), jnp.float32),
            pltpu.SemaphoreType.DMA,
            pltpu.SemaphoreType.DMA,
            pltpu.SemaphoreType.DMA,
        ],
    )
    def k(table_hbm, idx_hbm, out_hbm, idx_v, buf_v, sem_g, sem_w0, sem_w1):
        cid = lax.axis_index("c")
        sid = lax.axis_index("s")
        wid = sid * 2 + cid
        pltpu.sync_copy(idx_hbm.at[wid], idx_v)
        sem_w = [sem_w0, sem_w1]
        wr = [None, None]
        for gi, (g0, gsz) in enumerate(groups):
            b = gi % 2
            if wr[b] is not None:
                wr[b].wait()
            pltpu.async_copy(table_hbm.at[idx_v.at[pl.ds(g0, gsz)]],
                             buf_v.at[b, pl.ds(0, gsz)], sem_g).wait()
            wr[b] = pltpu.async_copy(buf_v.at[b, pl.ds(0, gsz)],
                                     out_hbm.at[wid, pl.ds(g0, gsz)], sem_w[b])
        for w in wr:
            if w is not None:
                w.wait()

    return k(table, idx2)


def _gather_rows(table, idx_pad):
    B = idx_pad.shape[0]
    out = _sc_gather(table, idx_pad.reshape(NW, B // NW))
    return out.reshape(B, table.shape[1])


def _sc_scatter_add(payload3, idx2, zeros, S, pd):
    _, R, D = payload3.shape
    rpt = S // 16
    GR = _GRP * CH
    groups = [(g, min(GR, R - g)) for g in range(0, R, GR)]
    phases = [(c0, pd) for c0 in range(0, D, pd)]
    mesh = plsc.VectorSubcoreMesh(core_axis_name="c", subcore_axis_name="s")

    @functools.partial(
        pl.kernel,
        out_type=jax.ShapeDtypeStruct((2, S, D), jnp.float32),
        mesh=mesh,
        compiler_params=pltpu.CompilerParams(use_tc_tiling_on_sc=False),
        scratch_types=[
            pltpu.VMEM((R,), jnp.int32),
            pltpu.VMEM((2, GR, pd), jnp.float32),
            pltpu.VMEM_SHARED((S, pd), jnp.float32),
            pltpu.SemaphoreType.DMA,
            pltpu.SemaphoreType.DMA,
            pltpu.SemaphoreType.DMA,
        ],
    )
    def k(pay_hbm, idx_hbm, z_hbm, out_hbm, idx_v, buf_v, acc_sh,
          sem_a, sem_l0, sem_l1):
        cid = lax.axis_index("c")
        sid = lax.axis_index("s")
        wid = sid * 2 + cid
        pltpu.sync_copy(idx_hbm.at[wid], idx_v)
        sem_l = [sem_l0, sem_l1]
        for c0, _ in phases:
            pltpu.sync_copy(z_hbm.at[pl.ds(sid * rpt, rpt)],
                            acc_sh.at[pl.ds(sid * rpt, rpt)])
            plsc.subcore_barrier()
            g0, gsz = groups[0]
            ld = [None, None]
            ld[0] = pltpu.async_copy(
                pay_hbm.at[wid, pl.ds(g0, gsz), pl.ds(c0, pd)],
                buf_v.at[0, pl.ds(0, gsz)], sem_l[0])
            for gi, (g0, gsz) in enumerate(groups):
                b = gi % 2
                ld[b].wait()
                if gi + 1 < len(groups):
                    n0, nsz = groups[gi + 1]
                    nb = (gi + 1) % 2
                    ld[nb] = pltpu.async_copy(
                        pay_hbm.at[wid, pl.ds(n0, nsz), pl.ds(c0, pd)],
                        buf_v.at[nb, pl.ds(0, nsz)], sem_l[nb])
                pltpu.async_copy(buf_v.at[b, pl.ds(0, gsz)],
                                 acc_sh.at[idx_v.at[pl.ds(g0, gsz)]],
                                 sem_a, add=True).wait()
            plsc.subcore_barrier()
            pltpu.sync_copy(acc_sh.at[pl.ds(sid * rpt, rpt)],
                            out_hbm.at[cid, pl.ds(sid * rpt, rpt), pl.ds(c0, pd)])
            plsc.subcore_barrier()

    return k(payload3, idx2, zeros)


def _scatter_add(payload, idx_pad, nseg):
    B, D = payload.shape
    pay3 = payload.reshape(NW, B // NW, D)
    pd = D if nseg * D <= 600 * 1024 else D // 2
    zeros = jnp.zeros((nseg, pd), jnp.float32)
    return _sc_scatter_add(pay3, idx_pad.reshape(NW, B // NW), zeros, nseg, pd)


def kernel(node_attr, edge_index, edge_attr, edge_length, ee_index, ee_angle,
           W_proj, b_proj, W_bond, b_bond, W_edgefn, b_edgefn, W_bu, b_bu,
           W_au, b_au, gnn_bias, gru_Wih, gru_Whh, gru_bih, gru_bhh,
           s2s_Wih0, s2s_Whh0, s2s_bih0, s2s_bhh0,
           s2s_Wih1, s2s_Whh1, s2s_bih1, s2s_bhh1, W_sp, b_sp, prelu_a):
    f32 = jnp.float32
    src = edge_index[0].astype(jnp.int32)
    dst = edge_index[1].astype(jnp.int32)
    ee_src = ee_index[0].astype(jnp.int32)
    ee_dst = ee_index[1].astype(jnp.int32)

    na_pad = jnp.pad(node_attr, ((0, NP - N), (0, 112 - D_NODE)))
    wp_pad = jnp.pad(W_proj, ((0, 112 - D_NODE), (0, 0)))
    w2f = W_edgefn.reshape(16, H, H).transpose(1, 0, 2).reshape(H, 16 * H)
    bline = b_edgefn.reshape(H, H)
    wbf = W_bond.reshape(16, H, H).transpose(1, 0, 2).reshape(H, 16 * H)
    bbond = b_bond.reshape(H, H)

    ee_src_p = jnp.pad(ee_src, (0, E2P - E2))
    ee_dst_p = jnp.pad(ee_dst, (0, E2P - E2))
    dst_p = jnp.pad(dst, (0, EP - E))
    src_p = jnp.pad(src, (0, EP - E))
    angT = jnp.pad(ee_angle, (0, E2P - E2))[None, :]
    eaT = jnp.pad(edge_attr, ((0, EP - E), (0, 0))).T
    elT = jnp.pad(edge_length, (0, EP - E))[None, :]

    src_bits = lax.bitcast_convert_type(src, f32)
    table16 = jnp.zeros((E, 16), f32)
    table16 = table16.at[:, 0].set(edge_length)
    table16 = table16.at[:, 1].set(src_bits)

    g2 = _gather_rows(table16, ee_src_p)
    src2_p = lax.bitcast_convert_type(g2[:, 1], jnp.int32)
    el2T = g2[:, 0][None, :]
    idxg = jnp.concatenate([src_p, src2_p])

    nf = _node_proj(na_pad, wp_pad, b_proj[None])

    x = nf
    h_gru = nf
    for _ in range(STEPS):
        g = _gather_rows(x, idxg)
        bm = g[:EP]
        bm2 = g[EP:]
        out_line = _line_msg(angT, el2T, bm2, W_au, b_au[:, None], w2f, bline)
        ap = _scatter_add(out_line, ee_dst_p, EP)
        m = _bond_msg(eaT, elT, bm, ap, W_bu, b_bu[:, None], wbf, bbond)
        npart = _scatter_add(m, dst_p, NP)
        x = _gru_step(npart, h_gru, gnn_bias[None], gru_Wih, gru_Whh,
                      gru_bih[None], gru_bhh[None])
        h_gru = x

    return _set2set(x, nf, s2s_Wih0, s2s_Whh0, s2s_bih0[None], s2s_bhh0[None],
                    s2s_Wih1, s2s_Whh1, s2s_bih1[None], s2s_bhh1[None],
                    W_sp, b_sp[None], prelu_a.reshape(1, 1))


# speedup vs baseline: 23.7127x; 1.0667x over previous
import functools

import jax
import jax.numpy as jnp
from jax import lax
from jax.experimental import pallas as pl
from jax.experimental.pallas import tpu as pltpu
from jax.experimental.pallas import tpu_sc as plsc

N = 20000
E = 50000
E2 = 100000
D_NODE = 110
D_EATTR = 8
K = 8
H = 32
D_HID = 4096
STEPS = 2
POOL_ITERS = 3

CUTOFF = 5.0
GAMMA = 10.0
CENTERS = [CUTOFF * i / (K - 1) for i in range(K)]

NW = 32
CH = 128
C_E = 13
C_E2 = 25
C_G = C_E + C_E2
EP = NW * C_E * CH
E2P = NW * C_E2 * CH
BG = NW * C_G * CH
NP = 20480
BLK = 2048

_INTERP = False


def _rbf_col(d, k):
    return jnp.exp(-GAMMA * (d - CENTERS[k]) ** 2)


def _k0_body(na_ref, wp_ref, bp_ref, o_ref):
    i = pl.program_id(0)
    x = jnp.maximum(jnp.dot(na_ref[...], wp_ref[...],
                            preferred_element_type=jnp.float32) + bp_ref[...], 0.0)
    rows = i * BLK + lax.broadcasted_iota(jnp.int32, (BLK, 1), 0)
    o_ref[...] = jnp.where(rows < N, x, 0.0)


def _node_proj(na_pad, wp_pad, bp):
    return pl.pallas_call(
        _k0_body,
        grid=(NP // BLK,),
        in_specs=[
            pl.BlockSpec((BLK, 112), lambda i: (i, 0)),
            pl.BlockSpec((112, H), lambda i: (0, 0)),
            pl.BlockSpec((1, H), lambda i: (0, 0)),
        ],
        out_specs=pl.BlockSpec((BLK, H), lambda i: (i, 0)),
        out_shape=jax.ShapeDtypeStruct((NP, H), jnp.float32),
        interpret=_INTERP,
    )(na_pad, wp_pad, bp)


def _dotT(a, b):
    return lax.dot_general(a, b, (((0,), (1,)), ((), ())),
                           preferred_element_type=jnp.float32)


def _dot00(a, b):
    return lax.dot_general(a, b, (((0,), (0,)), ((), ())),
                           preferred_element_type=jnp.float32)


def _kb_body(angT_ref, elT_ref, bm2_ref, wau_ref, bauT_ref, w2f_ref, bl_ref, o_ref):
    i = pl.program_id(0)
    hlT = jnp.maximum(_dotT(wau_ref[...], bm2_ref[...]) + bauT_ref[...], 0.0)
    yT = _dot00(w2f_ref[...], hlT)
    accT = _dot00(bl_ref[...], hlT)
    angT = angT_ref[...]
    elT = elT_ref[...]
    for k in range(K):
        accT += yT[k * H:(k + 1) * H] * _rbf_col(angT, k)
    for k in range(K):
        accT += yT[(K + k) * H:(K + k + 1) * H] * _rbf_col(elT, k)
    rows = i * BLK + lax.broadcasted_iota(jnp.int32, (BLK, 1), 0)
    o_ref[...] = jnp.where(rows < E2, accT.T, 0.0)


def _line_msg(angT, el2T, bm2, wau, bauT, w2f, bline):
    return pl.pallas_call(
        _kb_body,
        grid=(E2P // BLK,),
        in_specs=[
            pl.BlockSpec((1, BLK), lambda i: (0, i)),
            pl.BlockSpec((1, BLK), lambda i: (0, i)),
            pl.BlockSpec((BLK, H), lambda i: (EP // BLK + i, 0)),
            pl.BlockSpec((H, H), lambda i: (0, 0)),
            pl.BlockSpec((H, 1), lambda i: (0, 0)),
            pl.BlockSpec((H, 16 * H), lambda i: (0, 0)),
            pl.BlockSpec((H, H), lambda i: (0, 0)),
        ],
        out_specs=pl.BlockSpec((BLK, H), lambda i: (i, 0)),
        out_shape=jax.ShapeDtypeStruct((E2P, H), jnp.float32),
        interpret=_INTERP,
    )(angT, el2T, bm2, wau, bauT, w2f, bline)


def _kc_body(eaT_ref, elT_ref, bm_ref, ap_ref, wbu_ref, bbuT_ref, wbf_ref, bb_ref, o_ref):
    i = pl.program_id(0)
    hbT = jnp.maximum(_dotT(wbu_ref[...], bm_ref[...]) + bbuT_ref[...], 0.0)
    hbT = hbT + (ap_ref[0] + ap_ref[1]).T
    yT = _dot00(wbf_ref[...], hbT)
    accT = _dot00(bb_ref[...], hbT)
    eaT = eaT_ref[...]
    elT = elT_ref[...]
    for k in range(D_EATTR):
        accT += yT[k * H:(k + 1) * H] * eaT[k:k + 1]
    for k in range(K):
        accT += yT[(D_EATTR + k) * H:(D_EATTR + k + 1) * H] * _rbf_col(elT, k)
    rows = i * BLK + lax.broadcasted_iota(jnp.int32, (BLK, 1), 0)
    o_ref[...] = jnp.where(rows < E, accT.T, 0.0)


def _bond_msg(eaT, elT, bm, ap, wbu, bbuT, wbf, bbond):
    return pl.pallas_call(
        _kc_body,
        grid=(EP // BLK,),
        in_specs=[
            pl.BlockSpec((D_EATTR, BLK), lambda i: (0, i)),
            pl.BlockSpec((1, BLK), lambda i: (0, i)),
            pl.BlockSpec((BLK, H), lambda i: (i, 0)),
            pl.BlockSpec((2, BLK, H), lambda i: (0, i, 0)),
            pl.BlockSpec((H, H), lambda i: (0, 0)),
            pl.BlockSpec((H, 1), lambda i: (0, 0)),
            pl.BlockSpec((H, 16 * H), lambda i: (0, 0)),
            pl.BlockSpec((H, H), lambda i: (0, 0)),
        ],
        out_specs=pl.BlockSpec((BLK, H), lambda i: (i, 0)),
        out_shape=jax.ShapeDtypeStruct((EP, H), jnp.float32),
        interpret=_INTERP,
    )(eaT, elT, bm, ap, wbu, bbuT, wbf, bbond)


def _kd_body(np_ref, h_ref, gb_ref, wih_ref, whh_ref, bih_ref, bhh_ref, o_ref):
    x = jnp.maximum(np_ref[0] + np_ref[1] + gb_ref[...], 0.0)
    h = h_ref[...]
    gi = jnp.dot(x, wih_ref[...].T, preferred_element_type=jnp.float32) + bih_ref[...]
    gh = jnp.dot(h, whh_ref[...].T, preferred_element_type=jnp.float32) + bhh_ref[...]
    r = jax.nn.sigmoid(gi[:, :H] + gh[:, :H])
    z = jax.nn.sigmoid(gi[:, H:2 * H] + gh[:, H:2 * H])
    n = jnp.tanh(gi[:, 2 * H:] + r * gh[:, 2 * H:])
    o_ref[...] = (1.0 - z) * n + z * h


def _gru_step(npart, h_gru, gnn_bias, wih, whh, bih, bhh):
    return pl.pallas_call(
        _kd_body,
        grid=(NP // BLK,),
        in_specs=[
            pl.BlockSpec((2, BLK, H), lambda i: (0, i, 0)),
            pl.BlockSpec((BLK, H), lambda i: (i, 0)),
            pl.BlockSpec((1, H), lambda i: (0, 0)),
            pl.BlockSpec((3 * H, H), lambda i: (0, 0)),
            pl.BlockSpec((3 * H, H), lambda i: (0, 0)),
            pl.BlockSpec((1, 3 * H), lambda i: (0, 0)),
            pl.BlockSpec((1, 3 * H), lambda i: (0, 0)),
        ],
        out_specs=pl.BlockSpec((BLK, H), lambda i: (i, 0)),
        out_shape=jax.ShapeDtypeStruct((NP, H), jnp.float32),
        interpret=_INTERP,
    )(npart, h_gru, gnn_bias, wih, whh, bih, bhh)


def _lstm(x, h, c, wih, whh, bih, bhh):
    d = h.shape[-1]
    g = (jnp.dot(x, wih.T, preferred_element_type=jnp.float32) + bih
         + jnp.dot(h, whh.T, preferred_element_type=jnp.float32) + bhh)
    i = jax.nn.sigmoid(g[:, :d])
    f = jax.nn.sigmoid(g[:, d:2 * d])
    gg = jnp.tanh(g[:, 2 * d:3 * d])
    o = jax.nn.sigmoid(g[:, 3 * d:])
    c2 = f * c + i * gg
    return o * jnp.tanh(c2), c2


def _kz_body(x_ref, nf_ref, wih0_ref, whh0_ref, bih0_ref, bhh0_ref,
             wih1_ref, whh1_ref, bih1_ref, bhh1_ref, wsp_ref, bsp_ref, pa_ref,
             o_ref):
    na = jnp.concatenate([x_ref[...], nf_ref[...]], axis=1)
    rows = lax.broadcasted_iota(jnp.int32, (NP, 1), 0)
    valid = rows < N
    d = 2 * H
    q_star = jnp.zeros((1, 2 * d), jnp.float32)
    h0 = jnp.zeros((1, d), jnp.float32)
    c0 = jnp.zeros((1, d), jnp.float32)
    h1 = jnp.zeros((1, d), jnp.float32)
    c1 = jnp.zeros((1, d), jnp.float32)
    for _ in range(POOL_ITERS):
        h0, c0 = _lstm(q_star, h0, c0, wih0_ref[...], whh0_ref[...],
                       bih0_ref[...], bhh0_ref[...])
        h1, c1 = _lstm(h0, h1, c1, wih1_ref[...], whh1_ref[...],
                       bih1_ref[...], bhh1_ref[...])
        q = h1
        e = jnp.sum(na * q, axis=-1, keepdims=True)
        e = jnp.where(valid, e, -1e30)
        m = jnp.max(e, axis=0, keepdims=True)
        p = jnp.where(valid, jnp.exp(e - m), 0.0)
        alpha = p / jnp.sum(p, axis=0, keepdims=True)
        readout = jnp.sum(na * alpha, axis=0, keepdims=True)
        q_star = jnp.concatenate([q, readout], axis=-1)
    y = jnp.dot(q_star, wsp_ref[...], preferred_element_type=jnp.float32) + bsp_ref[...]
    o_ref[...] = jnp.where(y >= 0.0, y, pa_ref[...] * y)


def _set2set(x, nf, wih0, whh0, bih0, bhh0, wih1, whh1, bih1, bhh1, wsp, bsp, pa):
    full = lambda s: pl.BlockSpec(s, lambda: tuple(0 for _ in s))
    return pl.pallas_call(
        _kz_body,
        in_specs=[
            full((NP, H)), full((NP, H)),
            full((4 * 2 * H, 4 * H)), full((4 * 2 * H, 2 * H)),
            full((1, 4 * 2 * H)), full((1, 4 * 2 * H)),
            full((4 * 2 * H, 2 * H)), full((4 * 2 * H, 2 * H)),
            full((1, 4 * 2 * H)), full((1, 4 * 2 * H)),
            full((4 * H, D_HID)), full((1, D_HID)), full((1, 1)),
        ],
        out_specs=full((1, D_HID)),
        out_shape=jax.ShapeDtypeStruct((1, D_HID), jnp.float32),
        interpret=_INTERP,
    )(x, nf, wih0, whh0, bih0, bhh0, wih1, whh1, bih1, bhh1, wsp, bsp, pa)


_GRP = 13


def _sc_gather(table, idx2):
    _, R = idx2.shape
    D = table.shape[1]
    GR = _GRP * CH
    groups = [(g, min(GR, R - g)) for g in range(0, R, GR)]
    mesh = plsc.VectorSubcoreMesh(core_axis_name="c", subcore_axis_name="s")

    @functools.partial(
        pl.kernel,
        out_type=jax.ShapeDtypeStruct((NW, R, D), jnp.float32),
        mesh=mesh,
        compiler_params=pltpu.CompilerParams(use_tc_tiling_on_sc=False),
        scratch_types=[
            pltpu.VMEM((R,), jnp.int32),
            pltpu.VMEM((2, GR, D), jnp.float32),
            pltpu.SemaphoreType.DMA,
            pltpu.SemaphoreType.DMA,
            pltpu.SemaphoreType.DMA,
        ],
    )
    def k(table_hbm, idx_hbm, out_hbm, idx_v, buf_v, sem_g, sem_w0, sem_w1):
        cid = lax.axis_index("c")
        sid = lax.axis_index("s")
        wid = sid * 2 + cid
        pltpu.sync_copy(idx_hbm.at[wid], idx_v)
        sem_w = [sem_w0, sem_w1]
        wr = [None, None]
        for gi, (g0, gsz) in enumerate(groups):
            b = gi % 2
            if wr[b] is not None:
                wr[b].wait()
            pltpu.async_copy(table_hbm.at[idx_v.at[pl.ds(g0, gsz)]],
                             buf_v.at[b, pl.ds(0, gsz)], sem_g).wait()
            wr[b] = pltpu.async_copy(buf_v.at[b, pl.ds(0, gsz)],
                                     out_hbm.at[wid, pl.ds(g0, gsz)], sem_w[b])
        for w in wr:
            if w is not None:
                w.wait()

    return k(table, idx2)


def _gather_rows(table, idx_pad):
    B = idx_pad.shape[0]
    out = _sc_gather(table, idx_pad.reshape(NW, B // NW))
    return out.reshape(B, table.shape[1])


def _sc_scatter_add(payload3, idx2, zeros, S, pd):
    _, R, D = payload3.shape
    rpt = S // 16
    GR = _GRP * CH
    groups = [(g, min(GR, R - g)) for g in range(0, R, GR)]
    phases = [(c0, pd) for c0 in range(0, D, pd)]
    mesh = plsc.VectorSubcoreMesh(core_axis_name="c", subcore_axis_name="s")

    @functools.partial(
        pl.kernel,
        out_type=jax.ShapeDtypeStruct((2, S, D), jnp.float32),
        mesh=mesh,
        compiler_params=pltpu.CompilerParams(use_tc_tiling_on_sc=False),
        scratch_types=[
            pltpu.VMEM((R,), jnp.int32),
            pltpu.VMEM((2, GR, pd), jnp.float32),
            pltpu.VMEM_SHARED((S, pd), jnp.float32),
            pltpu.SemaphoreType.DMA,
            pltpu.SemaphoreType.DMA,
            pltpu.SemaphoreType.DMA,
        ],
    )
    def k(pay_hbm, idx_hbm, z_hbm, out_hbm, idx_v, buf_v, acc_sh,
          sem_a, sem_l0, sem_l1):
        cid = lax.axis_index("c")
        sid = lax.axis_index("s")
        wid = sid * 2 + cid
        pltpu.sync_copy(idx_hbm.at[wid], idx_v)
        sem_l = [sem_l0, sem_l1]
        for c0, _ in phases:
            pltpu.sync_copy(z_hbm.at[pl.ds(sid * rpt, rpt)],
                            acc_sh.at[pl.ds(sid * rpt, rpt)])
            plsc.subcore_barrier()
            g0, gsz = groups[0]
            ld = [None, None]
            ld[0] = pltpu.async_copy(
                pay_hbm.at[wid, pl.ds(g0, gsz), pl.ds(c0, pd)],
                buf_v.at[0, pl.ds(0, gsz)], sem_l[0])
            for gi, (g0, gsz) in enumerate(groups):
                b = gi % 2
                ld[b].wait()
                if gi + 1 < len(groups):
                    n0, nsz = groups[gi + 1]
                    nb = (gi + 1) % 2
                    ld[nb] = pltpu.async_copy(
                        pay_hbm.at[wid, pl.ds(n0, nsz), pl.ds(c0, pd)],
                        buf_v.at[nb, pl.ds(0, nsz)], sem_l[nb])
                pltpu.async_copy(buf_v.at[b, pl.ds(0, gsz)],
                                 acc_sh.at[idx_v.at[pl.ds(g0, gsz)]],
                                 sem_a, add=True).wait()
            plsc.subcore_barrier()
            pltpu.sync_copy(acc_sh.at[pl.ds(sid * rpt, rpt)],
                            out_hbm.at[cid, pl.ds(sid * rpt, rpt), pl.ds(c0, pd)])
            plsc.subcore_barrier()

    return k(payload3, idx2, zeros)


def _scatter_add(payload, idx_pad, nseg):
    B, D = payload.shape
    pay3 = payload.reshape(NW, B // NW, D)
    pd = D if nseg * D <= 600 * 1024 else D // 2
    zeros = jnp.zeros((nseg, pd), jnp.float32)
    return _sc_scatter_add(pay3, idx_pad.reshape(NW, B // NW), zeros, nseg, pd)


def kernel(node_attr, edge_index, edge_attr, edge_length, ee_index, ee_angle,
           W_proj, b_proj, W_bond, b_bond, W_edgefn, b_edgefn, W_bu, b_bu,
           W_au, b_au, gnn_bias, gru_Wih, gru_Whh, gru_bih, gru_bhh,
           s2s_Wih0, s2s_Whh0, s2s_bih0, s2s_bhh0,
           s2s_Wih1, s2s_Whh1, s2s_bih1, s2s_bhh1, W_sp, b_sp, prelu_a):
    f32 = jnp.float32
    src = edge_index[0].astype(jnp.int32)
    dst = edge_index[1].astype(jnp.int32)
    ee_src = ee_index[0].astype(jnp.int32)
    ee_dst = ee_index[1].astype(jnp.int32)

    na_pad = jnp.pad(node_attr, ((0, NP - N), (0, 112 - D_NODE)))
    wp_pad = jnp.pad(W_proj, ((0, 112 - D_NODE), (0, 0)))
    w2f = W_edgefn.reshape(16, H, H).transpose(1, 0, 2).reshape(H, 16 * H)
    bline = b_edgefn.reshape(H, H)
    wbf = W_bond.reshape(16, H, H).transpose(1, 0, 2).reshape(H, 16 * H)
    bbond = b_bond.reshape(H, H)

    ee_src_p = jnp.pad(ee_src, (0, E2P - E2))
    ee_dst_p = jnp.pad(ee_dst, (0, E2P - E2))
    dst_p = jnp.pad(dst, (0, EP - E))
    src_p = jnp.pad(src, (0, EP - E))
    angT = jnp.pad(ee_angle, (0, E2P - E2))[None, :]
    eaT = jnp.pad(edge_attr, ((0, EP - E), (0, 0))).T
    elT = jnp.pad(edge_length, (0, EP - E))[None, :]

    src_bits = lax.bitcast_convert_type(src, f32)
    table16 = jnp.zeros((E, 16), f32)
    table16 = table16.at[:, 0].set(edge_length)
    table16 = table16.at[:, 1].set(src_bits)

    g2 = _gather_rows(table16, ee_src_p)
    src2_p = lax.bitcast_convert_type(g2[:, 1], jnp.int32)
    el2T = g2[:, 0][None, :]
    idxg = jnp.concatenate([src_p, src2_p])

    nf = _node_proj(na_pad, wp_pad, b_proj[None])

    x = nf
    h_gru = nf
    for _ in range(STEPS):
        g = _gather_rows(x, idxg)
        out_line = _line_msg(angT, el2T, g, W_au, b_au[:, None], w2f, bline)
        ap = _scatter_add(out_line, ee_dst_p, EP)
        m = _bond_msg(eaT, elT, g, ap, W_bu, b_bu[:, None], wbf, bbond)
        npart = _scatter_add(m, dst_p, NP)
        x = _gru_step(npart, h_gru, gnn_bias[None], gru_Wih, gru_Whh,
                      gru_bih[None], gru_bhh[None])
        h_gru = x

    return _set2set(x, nf, s2s_Wih0, s2s_Whh0, s2s_bih0[None], s2s_bhh0[None],
                    s2s_Wih1, s2s_Whh1, s2s_bih1[None], s2s_bhh1[None],
                    W_sp, b_sp[None], prelu_a.reshape(1, 1))


# speedup vs baseline: 24.3215x; 1.0257x over previous
import functools

import jax
import jax.numpy as jnp
from jax import lax
from jax.experimental import pallas as pl
from jax.experimental.pallas import tpu as pltpu
from jax.experimental.pallas import tpu_sc as plsc

N = 20000
E = 50000
E2 = 100000
D_NODE = 110
D_EATTR = 8
K = 8
H = 32
D_HID = 4096
STEPS = 2
POOL_ITERS = 3

CUTOFF = 5.0
GAMMA = 10.0
CENTERS = [CUTOFF * i / (K - 1) for i in range(K)]

NW = 32
CH = 128
C_E = 13
C_E2 = 25
C_G = C_E + C_E2
EP = NW * C_E * CH
E2P = NW * C_E2 * CH
BG = NW * C_G * CH
NP = 20480
BLK = 2048

_INTERP = False


def _rbf_col(d, k):
    return jnp.exp(-GAMMA * (d - CENTERS[k]) ** 2)


def _k0_body(na_ref, wp_ref, bp_ref, o_ref):
    i = pl.program_id(0)
    x = jnp.maximum(jnp.dot(na_ref[...], wp_ref[...],
                            preferred_element_type=jnp.float32) + bp_ref[...], 0.0)
    rows = i * BLK + lax.broadcasted_iota(jnp.int32, (BLK, 1), 0)
    o_ref[...] = jnp.where(rows < N, x, 0.0)


def _node_proj(na_pad, wp_pad, bp):
    return pl.pallas_call(
        _k0_body,
        grid=(NP // BLK,),
        in_specs=[
            pl.BlockSpec((BLK, 112), lambda i: (i, 0)),
            pl.BlockSpec((112, H), lambda i: (0, 0)),
            pl.BlockSpec((1, H), lambda i: (0, 0)),
        ],
        out_specs=pl.BlockSpec((BLK, H), lambda i: (i, 0)),
        out_shape=jax.ShapeDtypeStruct((NP, H), jnp.float32),
        interpret=_INTERP,
    )(na_pad, wp_pad, bp)


def _dotT(a, b):
    return lax.dot_general(a, b, (((0,), (1,)), ((), ())),
                           preferred_element_type=jnp.float32)


def _dot00(a, b):
    return lax.dot_general(a, b, (((0,), (0,)), ((), ())),
                           preferred_element_type=jnp.float32)


def _kb_body(angT_ref, elT_ref, bm2_ref, wau_ref, bauT_ref, w2f_ref, bl_ref, o_ref):
    i = pl.program_id(0)
    hlT = jnp.maximum(_dotT(wau_ref[...], bm2_ref[...]) + bauT_ref[...], 0.0)
    yT = _dot00(w2f_ref[...], hlT)
    accT = _dot00(bl_ref[...], hlT)
    angT = angT_ref[...]
    elT = elT_ref[...]
    for k in range(K):
        accT += yT[k * H:(k + 1) * H] * _rbf_col(angT, k)
    for k in range(K):
        accT += yT[(K + k) * H:(K + k + 1) * H] * _rbf_col(elT, k)
    rows = i * BLK + lax.broadcasted_iota(jnp.int32, (BLK, 1), 0)
    o_ref[...] = jnp.where(rows < E2, accT.T, 0.0)


def _line_msg(angT, el2T, bm2, wau, bauT, w2f, bline):
    return pl.pallas_call(
        _kb_body,
        grid=(E2P // BLK,),
        in_specs=[
            pl.BlockSpec((1, BLK), lambda i: (0, i)),
            pl.BlockSpec((1, BLK), lambda i: (0, i)),
            pl.BlockSpec((BLK, H), lambda i: (EP // BLK + i, 0)),
            pl.BlockSpec((H, H), lambda i: (0, 0)),
            pl.BlockSpec((H, 1), lambda i: (0, 0)),
            pl.BlockSpec((H, 16 * H), lambda i: (0, 0)),
            pl.BlockSpec((H, H), lambda i: (0, 0)),
        ],
        out_specs=pl.BlockSpec((BLK, H), lambda i: (i, 0)),
        out_shape=jax.ShapeDtypeStruct((E2P, H), jnp.float32),
        interpret=_INTERP,
    )(angT, el2T, bm2, wau, bauT, w2f, bline)


def _kc_body(eaT_ref, elT_ref, bm_ref, ap_ref, wbu_ref, bbuT_ref, wbf_ref, bb_ref, o_ref):
    i = pl.program_id(0)
    hbT = jnp.maximum(_dotT(wbu_ref[...], bm_ref[...]) + bbuT_ref[...], 0.0)
    hbT = hbT + (ap_ref[0] + ap_ref[1]).T
    yT = _dot00(wbf_ref[...], hbT)
    accT = _dot00(bb_ref[...], hbT)
    eaT = eaT_ref[...]
    elT = elT_ref[...]
    for k in range(D_EATTR):
        accT += yT[k * H:(k + 1) * H] * eaT[k:k + 1]
    for k in range(K):
        accT += yT[(D_EATTR + k) * H:(D_EATTR + k + 1) * H] * _rbf_col(elT, k)
    rows = i * BLK + lax.broadcasted_iota(jnp.int32, (BLK, 1), 0)
    o_ref[...] = jnp.where(rows < E, accT.T, 0.0)


def _bond_msg(eaT, elT, bm, ap, wbu, bbuT, wbf, bbond):
    return pl.pallas_call(
        _kc_body,
        grid=(EP // BLK,),
        in_specs=[
            pl.BlockSpec((D_EATTR, BLK), lambda i: (0, i)),
            pl.BlockSpec((1, BLK), lambda i: (0, i)),
            pl.BlockSpec((BLK, H), lambda i: (i, 0)),
            pl.BlockSpec((2, BLK, H), lambda i: (0, i, 0)),
            pl.BlockSpec((H, H), lambda i: (0, 0)),
            pl.BlockSpec((H, 1), lambda i: (0, 0)),
            pl.BlockSpec((H, 16 * H), lambda i: (0, 0)),
            pl.BlockSpec((H, H), lambda i: (0, 0)),
        ],
        out_specs=pl.BlockSpec((BLK, H), lambda i: (i, 0)),
        out_shape=jax.ShapeDtypeStruct((EP, H), jnp.float32),
        interpret=_INTERP,
    )(eaT, elT, bm, ap, wbu, bbuT, wbf, bbond)


def _kd_body(np_ref, h_ref, gb_ref, wih_ref, whh_ref, bih_ref, bhh_ref, o_ref):
    x = jnp.maximum(np_ref[0] + np_ref[1] + gb_ref[...], 0.0)
    h = h_ref[...]
    gi = jnp.dot(x, wih_ref[...].T, preferred_element_type=jnp.float32) + bih_ref[...]
    gh = jnp.dot(h, whh_ref[...].T, preferred_element_type=jnp.float32) + bhh_ref[...]
    r = jax.nn.sigmoid(gi[:, :H] + gh[:, :H])
    z = jax.nn.sigmoid(gi[:, H:2 * H] + gh[:, H:2 * H])
    n = jnp.tanh(gi[:, 2 * H:] + r * gh[:, 2 * H:])
    o_ref[...] = (1.0 - z) * n + z * h


def _gru_step(npart, h_gru, gnn_bias, wih, whh, bih, bhh):
    return pl.pallas_call(
        _kd_body,
        grid=(NP // BLK,),
        in_specs=[
            pl.BlockSpec((2, BLK, H), lambda i: (0, i, 0)),
            pl.BlockSpec((BLK, H), lambda i: (i, 0)),
            pl.BlockSpec((1, H), lambda i: (0, 0)),
            pl.BlockSpec((3 * H, H), lambda i: (0, 0)),
            pl.BlockSpec((3 * H, H), lambda i: (0, 0)),
            pl.BlockSpec((1, 3 * H), lambda i: (0, 0)),
            pl.BlockSpec((1, 3 * H), lambda i: (0, 0)),
        ],
        out_specs=pl.BlockSpec((BLK, H), lambda i: (i, 0)),
        out_shape=jax.ShapeDtypeStruct((NP, H), jnp.float32),
        interpret=_INTERP,
    )(npart, h_gru, gnn_bias, wih, whh, bih, bhh)


def _lstm(x, h, c, wih, whh, bih, bhh):
    d = h.shape[-1]
    g = (jnp.dot(x, wih.T, preferred_element_type=jnp.float32) + bih
         + jnp.dot(h, whh.T, preferred_element_type=jnp.float32) + bhh)
    i = jax.nn.sigmoid(g[:, :d])
    f = jax.nn.sigmoid(g[:, d:2 * d])
    gg = jnp.tanh(g[:, 2 * d:3 * d])
    o = jax.nn.sigmoid(g[:, 3 * d:])
    c2 = f * c + i * gg
    return o * jnp.tanh(c2), c2


def _kz_body(x_ref, nf_ref, wih0_ref, whh0_ref, bih0_ref, bhh0_ref,
             wih1_ref, whh1_ref, bih1_ref, bhh1_ref, wsp_ref, bsp_ref, pa_ref,
             o_ref):
    naT = jnp.concatenate([x_ref[...].T, nf_ref[...].T], axis=0)
    cols = lax.broadcasted_iota(jnp.int32, (1, NP), 1)
    valid = cols < N
    d = 2 * H
    q_star = jnp.zeros((1, 2 * d), jnp.float32)
    h0 = jnp.zeros((1, d), jnp.float32)
    c0 = jnp.zeros((1, d), jnp.float32)
    h1 = jnp.zeros((1, d), jnp.float32)
    c1 = jnp.zeros((1, d), jnp.float32)
    for _ in range(POOL_ITERS):
        h0, c0 = _lstm(q_star, h0, c0, wih0_ref[...], whh0_ref[...],
                       bih0_ref[...], bhh0_ref[...])
        h1, c1 = _lstm(h0, h1, c1, wih1_ref[...], whh1_ref[...],
                       bih1_ref[...], bhh1_ref[...])
        q = h1
        e = jnp.dot(q, naT, preferred_element_type=jnp.float32)
        e = jnp.where(valid, e, -1e30)
        m = jnp.max(e, axis=1, keepdims=True)
        p = jnp.where(valid, jnp.exp(e - m), 0.0)
        alpha = p / jnp.sum(p, axis=1, keepdims=True)
        readout = lax.dot_general(alpha, naT, (((1,), (1,)), ((), ())),
                                  preferred_element_type=jnp.float32)
        q_star = jnp.concatenate([q, readout], axis=-1)
    y = jnp.dot(q_star, wsp_ref[...], preferred_element_type=jnp.float32) + bsp_ref[...]
    o_ref[...] = jnp.where(y >= 0.0, y, pa_ref[...] * y)


def _set2set(x, nf, wih0, whh0, bih0, bhh0, wih1, whh1, bih1, bhh1, wsp, bsp, pa):
    full = lambda s: pl.BlockSpec(s, lambda: tuple(0 for _ in s))
    return pl.pallas_call(
        _kz_body,
        in_specs=[
            full((NP, H)), full((NP, H)),
            full((4 * 2 * H, 4 * H)), full((4 * 2 * H, 2 * H)),
            full((1, 4 * 2 * H)), full((1, 4 * 2 * H)),
            full((4 * 2 * H, 2 * H)), full((4 * 2 * H, 2 * H)),
            full((1, 4 * 2 * H)), full((1, 4 * 2 * H)),
            full((4 * H, D_HID)), full((1, D_HID)), full((1, 1)),
        ],
        out_specs=full((1, D_HID)),
        out_shape=jax.ShapeDtypeStruct((1, D_HID), jnp.float32),
        interpret=_INTERP,
    )(x, nf, wih0, whh0, bih0, bhh0, wih1, whh1, bih1, bhh1, wsp, bsp, pa)


_GRP = 13


def _sc_gather(table, idx2):
    _, R = idx2.shape
    D = table.shape[1]
    GR = _GRP * CH
    groups = [(g, min(GR, R - g)) for g in range(0, R, GR)]
    mesh = plsc.VectorSubcoreMesh(core_axis_name="c", subcore_axis_name="s")

    @functools.partial(
        pl.kernel,
        out_type=jax.ShapeDtypeStruct((NW, R, D), jnp.float32),
        mesh=mesh,
        compiler_params=pltpu.CompilerParams(use_tc_tiling_on_sc=False),
        scratch_types=[
            pltpu.VMEM((R,), jnp.int32),
            pltpu.VMEM((2, GR, D), jnp.float32),
            pltpu.SemaphoreType.DMA,
            pltpu.SemaphoreType.DMA,
            pltpu.SemaphoreType.DMA,
        ],
    )
    def k(table_hbm, idx_hbm, out_hbm, idx_v, buf_v, sem_g, sem_w0, sem_w1):
        cid = lax.axis_index("c")
        sid = lax.axis_index("s")
        wid = sid * 2 + cid
        pltpu.sync_copy(idx_hbm.at[wid], idx_v)
        sem_w = [sem_w0, sem_w1]
        wr = [None, None]
        for gi, (g0, gsz) in enumerate(groups):
            b = gi % 2
            if wr[b] is not None:
                wr[b].wait()
            pltpu.async_copy(table_hbm.at[idx_v.at[pl.ds(g0, gsz)]],
                             buf_v.at[b, pl.ds(0, gsz)], sem_g).wait()
            wr[b] = pltpu.async_copy(buf_v.at[b, pl.ds(0, gsz)],
                                     out_hbm.at[wid, pl.ds(g0, gsz)], sem_w[b])
        for w in wr:
            if w is not None:
                w.wait()

    return k(table, idx2)


def _gather_rows(table, idx_pad):
    B = idx_pad.shape[0]
    out = _sc_gather(table, idx_pad.reshape(NW, B // NW))
    return out.reshape(B, table.shape[1])


def _sc_scatter_add(payload3, idx2, zeros, S, pd):
    _, R, D = payload3.shape
    rpt = S // 16
    GR = _GRP * CH
    groups = [(g, min(GR, R - g)) for g in range(0, R, GR)]
    phases = [(c0, pd) for c0 in range(0, D, pd)]
    mesh = plsc.VectorSubcoreMesh(core_axis_name="c", subcore_axis_name="s")

    @functools.partial(
        pl.kernel,
        out_type=jax.ShapeDtypeStruct((2, S, D), jnp.float32),
        mesh=mesh,
        compiler_params=pltpu.CompilerParams(use_tc_tiling_on_sc=False),
        scratch_types=[
            pltpu.VMEM((R,), jnp.int32),
            pltpu.VMEM((2, GR, pd), jnp.float32),
            pltpu.VMEM_SHARED((S, pd), jnp.float32),
            pltpu.SemaphoreType.DMA,
            pltpu.SemaphoreType.DMA,
            pltpu.SemaphoreType.DMA,
        ],
    )
    def k(pay_hbm, idx_hbm, z_hbm, out_hbm, idx_v, buf_v, acc_sh,
          sem_a, sem_l0, sem_l1):
        cid = lax.axis_index("c")
        sid = lax.axis_index("s")
        wid = sid * 2 + cid
        pltpu.sync_copy(idx_hbm.at[wid], idx_v)
        sem_l = [sem_l0, sem_l1]
        for c0, _ in phases:
            pltpu.sync_copy(z_hbm.at[pl.ds(sid * rpt, rpt)],
                            acc_sh.at[pl.ds(sid * rpt, rpt)])
            plsc.subcore_barrier()
            g0, gsz = groups[0]
            ld = [None, None]
            ld[0] = pltpu.async_copy(
                pay_hbm.at[wid, pl.ds(g0, gsz), pl.ds(c0, pd)],
                buf_v.at[0, pl.ds(0, gsz)], sem_l[0])
            for gi, (g0, gsz) in enumerate(groups):
                b = gi % 2
                ld[b].wait()
                if gi + 1 < len(groups):
                    n0, nsz = groups[gi + 1]
                    nb = (gi + 1) % 2
                    ld[nb] = pltpu.async_copy(
                        pay_hbm.at[wid, pl.ds(n0, nsz), pl.ds(c0, pd)],
                        buf_v.at[nb, pl.ds(0, nsz)], sem_l[nb])
                pltpu.async_copy(buf_v.at[b, pl.ds(0, gsz)],
                                 acc_sh.at[idx_v.at[pl.ds(g0, gsz)]],
                                 sem_a, add=True).wait()
            plsc.subcore_barrier()
            pltpu.sync_copy(acc_sh.at[pl.ds(sid * rpt, rpt)],
                            out_hbm.at[cid, pl.ds(sid * rpt, rpt), pl.ds(c0, pd)])
            plsc.subcore_barrier()

    return k(payload3, idx2, zeros)


def _scatter_add(payload, idx_pad, nseg):
    B, D = payload.shape
    pay3 = payload.reshape(NW, B // NW, D)
    pd = D if nseg * D <= 600 * 1024 else D // 2
    zeros = jnp.zeros((nseg, pd), jnp.float32)
    return _sc_scatter_add(pay3, idx_pad.reshape(NW, B // NW), zeros, nseg, pd)


def kernel(node_attr, edge_index, edge_attr, edge_length, ee_index, ee_angle,
           W_proj, b_proj, W_bond, b_bond, W_edgefn, b_edgefn, W_bu, b_bu,
           W_au, b_au, gnn_bias, gru_Wih, gru_Whh, gru_bih, gru_bhh,
           s2s_Wih0, s2s_Whh0, s2s_bih0, s2s_bhh0,
           s2s_Wih1, s2s_Whh1, s2s_bih1, s2s_bhh1, W_sp, b_sp, prelu_a):
    f32 = jnp.float32
    src = edge_index[0].astype(jnp.int32)
    dst = edge_index[1].astype(jnp.int32)
    ee_src = ee_index[0].astype(jnp.int32)
    ee_dst = ee_index[1].astype(jnp.int32)

    na_pad = jnp.pad(node_attr, ((0, NP - N), (0, 112 - D_NODE)))
    wp_pad = jnp.pad(W_proj, ((0, 112 - D_NODE), (0, 0)))
    w2f = W_edgefn.reshape(16, H, H).transpose(1, 0, 2).reshape(H, 16 * H)
    bline = b_edgefn.reshape(H, H)
    wbf = W_bond.reshape(16, H, H).transpose(1, 0, 2).reshape(H, 16 * H)
    bbond = b_bond.reshape(H, H)

    ee_src_p = jnp.pad(ee_src, (0, E2P - E2))
    ee_dst_p = jnp.pad(ee_dst, (0, E2P - E2))
    dst_p = jnp.pad(dst, (0, EP - E))
    src_p = jnp.pad(src, (0, EP - E))
    angT = jnp.pad(ee_angle, (0, E2P - E2))[None, :]
    eaT = jnp.pad(edge_attr, ((0, EP - E), (0, 0))).T
    elT = jnp.pad(edge_length, (0, EP - E))[None, :]

    src_bits = lax.bitcast_convert_type(src, f32)
    table16 = jnp.zeros((E, 16), f32)
    table16 = table16.at[:, 0].set(edge_length)
    table16 = table16.at[:, 1].set(src_bits)

    g2 = _gather_rows(table16, ee_src_p)
    src2_p = lax.bitcast_convert_type(g2[:, 1], jnp.int32)
    el2T = g2[:, 0][None, :]
    idxg = jnp.concatenate([src_p, src2_p])

    nf = _node_proj(na_pad, wp_pad, b_proj[None])

    x = nf
    h_gru = nf
    for _ in range(STEPS):
        g = _gather_rows(x, idxg)
        out_line = _line_msg(angT, el2T, g, W_au, b_au[:, None], w2f, bline)
        ap = _scatter_add(out_line, ee_dst_p, EP)
        m = _bond_msg(eaT, elT, g, ap, W_bu, b_bu[:, None], wbf, bbond)
        npart = _scatter_add(m, dst_p, NP)
        x = _gru_step(npart, h_gru, gnn_bias[None], gru_Wih, gru_Whh,
                      gru_bih[None], gru_bhh[None])
        h_gru = x

    return _set2set(x, nf, s2s_Wih0, s2s_Whh0, s2s_bih0[None], s2s_bhh0[None],
                    s2s_Wih1, s2s_Whh1, s2s_bih1[None], s2s_bhh1[None],
                    W_sp, b_sp[None], prelu_a.reshape(1, 1))
